# Initial kernel scaffold; baseline (speedup 1.0000x reference)
import jax
import jax.numpy as jnp
from jax.experimental import pallas as pl


def kernel(xyz, xyz_batch_cnt, new_xyz, new_xyz_batch_cnt, features, w0_0, w0_1, w1_0, w1_1):
    raise NotImplementedError("write your pallas kernel here")


# speedup vs baseline: 26.9752x; 26.9752x over previous
import functools

import jax
import jax.numpy as jnp
from jax import lax
from jax.experimental import pallas as pl
from jax.experimental.pallas import tpu as pltpu
from jax.experimental.pallas import tpu_sc as plsc

_RADII = (0.8, 1.6)
_NSAMPLES = (16, 32)
_EPS = 1e-5

_NC = 2
_NSUB = 16
_NW = _NC * _NSUB
_L = 16
_RC = 128


def _splat(v, dtype=jnp.int32):
    return jnp.full((_L,), v, dtype=dtype)


def _mm(x, w, tile):
    R, K = x.shape
    C = w.shape[1]

    def body(x_ref, w_ref, o_ref):
        o_ref[...] = jnp.dot(x_ref[...], w_ref[...],
                             preferred_element_type=jnp.float32)

    return pl.pallas_call(
        body,
        grid=(R // tile,),
        in_specs=[pl.BlockSpec((tile, K), lambda i: (i, 0)),
                  pl.BlockSpec((K, C), lambda i: (0, 0))],
        out_specs=pl.BlockSpec((tile, C), lambda i: (i, 0)),
        out_shape=jax.ShapeDtypeStruct((R, C), jnp.float32),
    )(x, w)


def _sc_ball_gather(xs, ys, zs, qx, qy, qz, h0, h1, n_per, m_per):
    n_src = xs.shape[0]
    m = qx.shape[0]
    qw = m // _NW
    ns_max = _NSAMPLES[-1]
    mesh = plsc.VectorSubcoreMesh(core_axis_name="c", subcore_axis_name="s",
                                  num_cores=_NC, num_subcores=_NSUB)

    @functools.partial(
        pl.kernel,
        out_type=[
            jax.ShapeDtypeStruct((m * _NSAMPLES[0], 16), jnp.float32),
            jax.ShapeDtypeStruct((m * _NSAMPLES[1], 16), jnp.float32),
            jax.ShapeDtypeStruct((m,), jnp.int32),
            jax.ShapeDtypeStruct((m,), jnp.int32),
        ],
        mesh=mesh,
        compiler_params=pltpu.CompilerParams(needs_layout_passes=False,
                                             use_tc_tiling_on_sc=False),
        scratch_types=[
            pltpu.VMEM((n_per,), jnp.float32),
            pltpu.VMEM((n_per,), jnp.float32),
            pltpu.VMEM((n_per,), jnp.float32),
            pltpu.VMEM((qw,), jnp.float32),
            pltpu.VMEM((qw,), jnp.float32),
            pltpu.VMEM((qw,), jnp.float32),
            pltpu.VMEM((qw * ns_max,), jnp.int32),
            pltpu.VMEM((qw,), jnp.int32),
            pltpu.VMEM((_RC, 16), jnp.float32),
            pltpu.SemaphoreType.DMA,
        ],
    )
    def sc_kernel(xs_h, ys_h, zs_h, qx_h, qy_h, qz_h, h0_h, h1_h,
                  g0_h, g1_h, cnt0_h, cnt1_h,
                  xs_v, ys_v, zs_v, qx_v, qy_v, qz_v, idx_v, cnt_v, rows_v,
                  sem):
        wid = lax.axis_index("s") * _NC + lax.axis_index("c")
        qbase = wid * qw
        batch = qbase // m_per
        nbase = batch * n_per

        pltpu.sync_copy(xs_h.at[pl.ds(nbase, n_per)], xs_v)
        pltpu.sync_copy(ys_h.at[pl.ds(nbase, n_per)], ys_v)
        pltpu.sync_copy(zs_h.at[pl.ds(nbase, n_per)], zs_v)
        pltpu.sync_copy(qx_h.at[pl.ds(qbase, qw)], qx_v)
        pltpu.sync_copy(qy_h.at[pl.ds(qbase, qw)], qy_v)
        pltpu.sync_copy(qz_h.at[pl.ds(qbase, qw)], qz_v)

        lanes = lax.broadcasted_iota(jnp.int32, (_L,), 0)

        for scale, (radius, ns) in enumerate(zip(_RADII, _NSAMPLES)):
            r2 = radius * radius
            h_h = (h0_h, h1_h)[scale]
            g_h = (g0_h, g1_h)[scale]
            c_h = (cnt0_h, cnt1_h)[scale]

            def per_query(q, _, ns=ns, r2=r2):
                qi = _splat(q)
                qxv = plsc.load_gather(qx_v, [qi])
                qyv = plsc.load_gather(qy_v, [qi])
                qzv = plsc.load_gather(qz_v, [qi])

                def cond(carry):
                    j, cnt = carry
                    return jnp.logical_and(cnt < ns, j < n_per)

                def scan_chunk(carry):
                    j, cnt = carry
                    dx = xs_v[pl.ds(j, _L)] - qxv
                    dy = ys_v[pl.ds(j, _L)] - qyv
                    dz = zs_v[pl.ds(j, _L)] - qzv
                    d2 = dx * dx + dy * dy + dz * dz
                    within = d2 < r2
                    wi = within.astype(jnp.int32)
                    nhit = jnp.sum(wi)
                    pos = plsc.cumsum(wi) - wi + cnt
                    smask = jnp.logical_and(within, pos < ns)
                    vals = lanes + (j + nbase)
                    plsc.store_scatter(idx_v, [pos + q * ns], vals,
                                       mask=smask)
                    return (j + _L, jnp.minimum(cnt + nhit, ns))

                _, cnt = lax.while_loop(cond, scan_chunk,
                                        (jnp.int32(0), jnp.int32(0)))

                first = plsc.load_gather(idx_v, [_splat(q * ns)])
                firstv = jnp.where(_splat(cnt) > 0, first, _splat(nbase))
                cntv = _splat(cnt)
                for ch in range(ns // _L):
                    sl = pl.ds(q * ns + ch * _L, _L)
                    keep = (lanes + ch * _L) < cntv
                    idx_v[sl] = jnp.where(keep, idx_v[sl], firstv)
                plsc.store_scatter(cnt_v, [_splat(q)], cntv,
                                   mask=lanes == 0)
                return _

            lax.fori_loop(0, qw, per_query, 0)

            def per_chunk(c, _):
                cp = pltpu.async_copy(
                    h_h.at[idx_v.at[pl.ds(c * _RC, _RC)]], rows_v, sem)
                cp.wait()
                pltpu.sync_copy(rows_v,
                                g_h.at[pl.ds(qbase * ns + c * _RC, _RC)])
                return _

            lax.fori_loop(0, qw * ns // _RC, per_chunk, 0)
            pltpu.sync_copy(cnt_v, c_h.at[pl.ds(qbase, qw)])

    return sc_kernel(xs, ys, zs, qx, qy, qz, h0, h1)


def _stats1(g3, o, mask, tile):
    m, ns, c = g3.shape

    def body(g_ref, o_ref, m_ref, out_ref):
        y = (g_ref[...] - o_ref[...][:, None, :]) * m_ref[...][:, :, None]
        s = jnp.sum(y, axis=(0, 1))
        sq = jnp.sum(y * y, axis=(0, 1))
        blk = jnp.concatenate([s[None, :], sq[None, :]], axis=0)

        @pl.when(pl.program_id(0) == 0)
        def _init():
            out_ref[...] = jnp.zeros_like(out_ref)

        out_ref[...] += blk

    return pl.pallas_call(
        body,
        grid=(m // tile,),
        in_specs=[pl.BlockSpec((tile, ns, c), lambda i: (i, 0, 0)),
                  pl.BlockSpec((tile, c), lambda i: (i, 0)),
                  pl.BlockSpec((tile, 1), lambda i: (i, 0))],
        out_specs=pl.BlockSpec((2, c), lambda i: (0, 0)),
        out_shape=jax.ShapeDtypeStruct((2, c), jnp.float32),
    )(g3, o, mask)


def _norm1(g_ref, o_ref, m_ref, st1_ref, n1):
    mu1 = st1_ref[0, :] / n1
    var1 = st1_ref[1, :] / n1 - mu1 * mu1
    inv1 = lax.rsqrt(var1 + _EPS)
    y1 = (g_ref[...] - o_ref[...][:, None, :]) * m_ref[...][:, :, None]
    return jnp.maximum((y1 - mu1) * inv1, 0.0)


def _stats2(g3, o, mask, st1, w2t, tile):
    m, ns, c = g3.shape
    c2 = w2t.shape[1]
    n1 = float(m * ns)

    def body(g_ref, o_ref, m_ref, st1_ref, w_ref, out_ref):
        x1 = _norm1(g_ref, o_ref, m_ref, st1_ref, n1)
        y2 = jnp.dot(x1.reshape(tile * ns, c), w_ref[...],
                     preferred_element_type=jnp.float32)
        s = jnp.sum(y2, axis=0)
        sq = jnp.sum(y2 * y2, axis=0)
        blk = jnp.concatenate([s[None, :], sq[None, :]], axis=0)

        @pl.when(pl.program_id(0) == 0)
        def _init():
            out_ref[...] = jnp.zeros_like(out_ref)

        out_ref[...] += blk

    return pl.pallas_call(
        body,
        grid=(m // tile,),
        in_specs=[pl.BlockSpec((tile, ns, c), lambda i: (i, 0, 0)),
                  pl.BlockSpec((tile, c), lambda i: (i, 0)),
                  pl.BlockSpec((tile, 1), lambda i: (i, 0)),
                  pl.BlockSpec((2, c), lambda i: (0, 0)),
                  pl.BlockSpec((c, c2), lambda i: (0, 0))],
        out_specs=pl.BlockSpec((2, c2), lambda i: (0, 0)),
        out_shape=jax.ShapeDtypeStruct((2, c2), jnp.float32),
    )(g3, o, mask, st1, w2t)


def _final(g3, o, mask, st1, st2, w2t, tile):
    m, ns, c = g3.shape
    c2 = w2t.shape[1]
    n1 = float(m * ns)

    def body(g_ref, o_ref, m_ref, st1_ref, st2_ref, w_ref, out_ref):
        x1 = _norm1(g_ref, o_ref, m_ref, st1_ref, n1)
        y2 = jnp.dot(x1.reshape(tile * ns, c), w_ref[...],
                     preferred_element_type=jnp.float32)
        mu2 = st2_ref[0, :] / n1
        var2 = st2_ref[1, :] / n1 - mu2 * mu2
        inv2 = lax.rsqrt(var2 + _EPS)
        x2 = jnp.maximum((y2 - mu2) * inv2, 0.0)
        out_ref[...] = jnp.max(x2.reshape(tile, ns, c2), axis=1)

    return pl.pallas_call(
        body,
        grid=(m // tile,),
        in_specs=[pl.BlockSpec((tile, ns, c), lambda i: (i, 0, 0)),
                  pl.BlockSpec((tile, c), lambda i: (i, 0)),
                  pl.BlockSpec((tile, 1), lambda i: (i, 0)),
                  pl.BlockSpec((2, c), lambda i: (0, 0)),
                  pl.BlockSpec((2, c2), lambda i: (0, 0)),
                  pl.BlockSpec((c, c2), lambda i: (0, 0))],
        out_specs=pl.BlockSpec((tile, c2), lambda i: (i, 0)),
        out_shape=jax.ShapeDtypeStruct((m, c2), jnp.float32),
    )(g3, o, mask, st1, st2, w2t)


def kernel(xyz, xyz_batch_cnt, new_xyz, new_xyz_batch_cnt, features,
           w0_0, w0_1, w1_0, w1_1):
    B = xyz_batch_cnt.shape[0]
    n_per = xyz.shape[0] // B
    m_per = new_xyz.shape[0] // B
    m = new_xyz.shape[0]

    u = jnp.concatenate([xyz, features], axis=1)
    h0 = _mm(u, w0_0.T, 2048)
    h1 = _mm(u, w1_0.T, 2048)
    o0 = _mm(new_xyz, w0_0[:, :3].T, 1024)
    o1 = _mm(new_xyz, w1_0[:, :3].T, 1024)

    g0f, g1f, cnt0, cnt1 = _sc_ball_gather(
        xyz[:, 0], xyz[:, 1], xyz[:, 2],
        new_xyz[:, 0], new_xyz[:, 1], new_xyz[:, 2],
        h0, h1, n_per, m_per)

    outs = []
    for scale, (w2, gf, o, cnt) in enumerate(
            [(w0_1, g0f, o0, cnt0), (w1_1, g1f, o1, cnt1)]):
        ns = _NSAMPLES[scale]
        g3 = gf.reshape(m, ns, 16)
        maskf = (cnt > 0).astype(jnp.float32).reshape(m, 1)
        w2t = w2.T
        tile = 512
        st1 = _stats1(g3, o, maskf, tile)
        st2 = _stats2(g3, o, maskf, st1, w2t, tile)
        outs.append(_final(g3, o, maskf, st1, st2, w2t, tile))

    return (new_xyz, jnp.concatenate(outs, axis=1))


# speedup vs baseline: 27.5571x; 1.0216x over previous
import functools

import jax
import jax.numpy as jnp
from jax import lax
from jax.experimental import pallas as pl
from jax.experimental.pallas import tpu as pltpu
from jax.experimental.pallas import tpu_sc as plsc

_RADII = (0.8, 1.6)
_NSAMPLES = (16, 32)
_EPS = 1e-5

_NC = 2
_NSUB = 16
_NW = _NC * _NSUB
_L = 16
_RC = 128


def _splat(v, dtype=jnp.int32):
    return jnp.full((_L,), v, dtype=dtype)


def _precompute(u, new_xyz, w00t, w10t, wq0, wq1):
    n, ku = u.shape
    m, kq = new_xyz.shape
    grid = 16
    tn, tm = n // grid, m // grid

    def body(u_ref, q_ref, w00_ref, w10_ref, wq0_ref, wq1_ref,
             h0_ref, h1_ref, o0_ref, o1_ref):
        uu = u_ref[...]
        qq = q_ref[...]
        h0_ref[...] = jnp.dot(uu, w00_ref[...], preferred_element_type=jnp.float32)
        h1_ref[...] = jnp.dot(uu, w10_ref[...], preferred_element_type=jnp.float32)
        o0_ref[...] = jnp.dot(qq, wq0_ref[...], preferred_element_type=jnp.float32)
        o1_ref[...] = jnp.dot(qq, wq1_ref[...], preferred_element_type=jnp.float32)

    return pl.pallas_call(
        body,
        grid=(grid,),
        in_specs=[pl.BlockSpec((tn, ku), lambda i: (i, 0)),
                  pl.BlockSpec((tm, kq), lambda i: (i, 0)),
                  pl.BlockSpec((ku, 16), lambda i: (0, 0)),
                  pl.BlockSpec((ku, 16), lambda i: (0, 0)),
                  pl.BlockSpec((kq, 16), lambda i: (0, 0)),
                  pl.BlockSpec((kq, 16), lambda i: (0, 0))],
        out_specs=[pl.BlockSpec((tn, 16), lambda i: (i, 0)),
                   pl.BlockSpec((tn, 16), lambda i: (i, 0)),
                   pl.BlockSpec((tm, 16), lambda i: (i, 0)),
                   pl.BlockSpec((tm, 16), lambda i: (i, 0))],
        out_shape=[jax.ShapeDtypeStruct((n, 16), jnp.float32),
                   jax.ShapeDtypeStruct((n, 16), jnp.float32),
                   jax.ShapeDtypeStruct((m, 16), jnp.float32),
                   jax.ShapeDtypeStruct((m, 16), jnp.float32)],
    )(u, new_xyz, w00t, w10t, wq0, wq1)


def _sc_ball_gather(xs, ys, zs, qx, qy, qz, h0, h1, n_per, m_per):
    m = qx.shape[0]
    qw = m // _NW
    ns_max = _NSAMPLES[-1]
    mesh = plsc.VectorSubcoreMesh(core_axis_name="c", subcore_axis_name="s",
                                  num_cores=_NC, num_subcores=_NSUB)

    @functools.partial(
        pl.kernel,
        out_type=[
            jax.ShapeDtypeStruct((m * _NSAMPLES[0], 16), jnp.float32),
            jax.ShapeDtypeStruct((m * _NSAMPLES[1], 16), jnp.float32),
            jax.ShapeDtypeStruct((m,), jnp.int32),
            jax.ShapeDtypeStruct((m,), jnp.int32),
        ],
        mesh=mesh,
        compiler_params=pltpu.CompilerParams(needs_layout_passes=False,
                                             use_tc_tiling_on_sc=False),
        scratch_types=[
            pltpu.VMEM((n_per,), jnp.float32),
            pltpu.VMEM((n_per,), jnp.float32),
            pltpu.VMEM((n_per,), jnp.float32),
            pltpu.VMEM((qw,), jnp.float32),
            pltpu.VMEM((qw,), jnp.float32),
            pltpu.VMEM((qw,), jnp.float32),
            pltpu.VMEM((qw * ns_max,), jnp.int32),
            pltpu.VMEM((qw,), jnp.int32),
            pltpu.VMEM((_RC, 16), jnp.float32),
            pltpu.SemaphoreType.DMA,
        ],
    )
    def sc_kernel(xs_h, ys_h, zs_h, qx_h, qy_h, qz_h, h0_h, h1_h,
                  g0_h, g1_h, cnt0_h, cnt1_h,
                  xs_v, ys_v, zs_v, qx_v, qy_v, qz_v, idx_v, cnt_v, rows_v,
                  sem):
        wid = lax.axis_index("s") * _NC + lax.axis_index("c")
        qbase = wid * qw
        batch = qbase // m_per
        nbase = batch * n_per

        pltpu.sync_copy(xs_h.at[pl.ds(nbase, n_per)], xs_v)
        pltpu.sync_copy(ys_h.at[pl.ds(nbase, n_per)], ys_v)
        pltpu.sync_copy(zs_h.at[pl.ds(nbase, n_per)], zs_v)
        pltpu.sync_copy(qx_h.at[pl.ds(qbase, qw)], qx_v)
        pltpu.sync_copy(qy_h.at[pl.ds(qbase, qw)], qy_v)
        pltpu.sync_copy(qz_h.at[pl.ds(qbase, qw)], qz_v)

        lanes = lax.broadcasted_iota(jnp.int32, (_L,), 0)

        for scale, (radius, ns) in enumerate(zip(_RADII, _NSAMPLES)):
            r2 = radius * radius
            h_h = (h0_h, h1_h)[scale]
            g_h = (g0_h, g1_h)[scale]
            c_h = (cnt0_h, cnt1_h)[scale]

            def per_query(q, _, ns=ns, r2=r2):
                qi = _splat(q)
                qxv = plsc.load_gather(qx_v, [qi])
                qyv = plsc.load_gather(qy_v, [qi])
                qzv = plsc.load_gather(qz_v, [qi])

                def cond(carry):
                    j, cnt = carry
                    return jnp.logical_and(cnt < ns, j < n_per)

                def scan_chunk(carry):
                    j, cnt = carry
                    dx = xs_v[pl.ds(j, _L)] - qxv
                    dy = ys_v[pl.ds(j, _L)] - qyv
                    dz = zs_v[pl.ds(j, _L)] - qzv
                    d2 = dx * dx + dy * dy + dz * dz
                    within = d2 < r2
                    nhitv = plsc.all_reduce_population_count(within)
                    nhit = nhitv[0]

                    @pl.when(nhit > 0)
                    def _emit():
                        wi = within.astype(jnp.int32)
                        pos = plsc.cumsum(wi) - wi + cnt
                        smask = jnp.logical_and(within, pos < ns)
                        vals = lanes + (j + nbase)
                        plsc.store_scatter(idx_v, [pos + q * ns], vals,
                                           mask=smask)

                    return (j + _L, jnp.minimum(cnt + nhit, ns))

                _, cnt = lax.while_loop(cond, scan_chunk,
                                        (jnp.int32(0), jnp.int32(0)))

                first = plsc.load_gather(idx_v, [_splat(q * ns)])
                firstv = jnp.where(_splat(cnt) > 0, first, _splat(nbase))
                cntv = _splat(cnt)
                for ch in range(ns // _L):
                    sl = pl.ds(q * ns + ch * _L, _L)
                    keep = (lanes + ch * _L) < cntv
                    idx_v[sl] = jnp.where(keep, idx_v[sl], firstv)
                plsc.store_scatter(cnt_v, [_splat(q)], cntv,
                                   mask=lanes == 0)
                return _

            lax.fori_loop(0, qw, per_query, 0)

            def per_chunk(c, _):
                cp = pltpu.async_copy(
                    h_h.at[idx_v.at[pl.ds(c * _RC, _RC)]], rows_v, sem)
                cp.wait()
                pltpu.sync_copy(rows_v,
                                g_h.at[pl.ds(qbase * ns + c * _RC, _RC)])
                return _

            lax.fori_loop(0, qw * ns // _RC, per_chunk, 0)
            pltpu.sync_copy(cnt_v, c_h.at[pl.ds(qbase, qw)])

    return sc_kernel(xs, ys, zs, qx, qy, qz, h0, h1)


def _y1(g_ref, o_ref, m_ref):
    return (g_ref[...] - o_ref[...][:, None, :]) * m_ref[...][:, :, None]


def _acc_sums(x, out_ref, axes):
    s = jnp.sum(x, axis=axes)
    sq = jnp.sum(x * x, axis=axes)
    blk = jnp.concatenate([s[None, :], sq[None, :]], axis=0)

    @pl.when(pl.program_id(0) == 0)
    def _init():
        out_ref[...] = jnp.zeros_like(out_ref)

    out_ref[...] += blk


def _stats1(g0, o0, m0, g1, o1, m1, tile):
    m = o0.shape[0]
    ns0, ns1 = _NSAMPLES

    def body(g0_ref, o0_ref, m0_ref, g1_ref, o1_ref, m1_ref,
             st0_ref, st1_ref):
        _acc_sums(_y1(g0_ref, o0_ref, m0_ref), st0_ref, (0, 1))
        _acc_sums(_y1(g1_ref, o1_ref, m1_ref), st1_ref, (0, 1))

    return pl.pallas_call(
        body,
        grid=(m // tile,),
        in_specs=[pl.BlockSpec((tile, ns0, 16), lambda i: (i, 0, 0)),
                  pl.BlockSpec((tile, 16), lambda i: (i, 0)),
                  pl.BlockSpec((tile, 1), lambda i: (i, 0)),
                  pl.BlockSpec((tile, ns1, 16), lambda i: (i, 0, 0)),
                  pl.BlockSpec((tile, 16), lambda i: (i, 0)),
                  pl.BlockSpec((tile, 1), lambda i: (i, 0))],
        out_specs=[pl.BlockSpec((2, 16), lambda i: (0, 0)),
                   pl.BlockSpec((2, 16), lambda i: (0, 0))],
        out_shape=[jax.ShapeDtypeStruct((2, 16), jnp.float32),
                   jax.ShapeDtypeStruct((2, 16), jnp.float32)],
    )(g0, o0, m0, g1, o1, m1)


def _bn(st_ref, n):
    mu = st_ref[0, :] / n
    var = st_ref[1, :] / n - mu * mu
    return mu, lax.rsqrt(var + _EPS)


def _pass2(g0, o0, m0, st0, w0t, g1, o1, m1, st1, w1t, tile):
    m = o0.shape[0]
    ns0, ns1 = _NSAMPLES
    n0, n1 = float(m * ns0), float(m * ns1)
    c20, c21 = w0t.shape[1], w1t.shape[1]

    def one(g_ref, o_ref, m_ref, st_ref, w_ref, s2_ref, mx_ref, ns, nn):
        mu1, inv1 = _bn(st_ref, nn)
        x1 = jnp.maximum((_y1(g_ref, o_ref, m_ref) - mu1) * inv1, 0.0)
        c2 = w_ref.shape[1]
        y2 = jnp.dot(x1.reshape(tile * ns, 16), w_ref[...],
                     preferred_element_type=jnp.float32)
        _acc_sums(y2, s2_ref, (0,))
        mx_ref[...] = jnp.max(y2.reshape(tile, ns, c2), axis=1)

    def body(g0_ref, o0_ref, m0_ref, st0_ref, w0_ref,
             g1_ref, o1_ref, m1_ref, st1_ref, w1_ref,
             s20_ref, s21_ref, mx0_ref, mx1_ref):
        one(g0_ref, o0_ref, m0_ref, st0_ref, w0_ref, s20_ref, mx0_ref,
            ns0, n0)
        one(g1_ref, o1_ref, m1_ref, st1_ref, w1_ref, s21_ref, mx1_ref,
            ns1, n1)

    return pl.pallas_call(
        body,
        grid=(m // tile,),
        in_specs=[pl.BlockSpec((tile, ns0, 16), lambda i: (i, 0, 0)),
                  pl.BlockSpec((tile, 16), lambda i: (i, 0)),
                  pl.BlockSpec((tile, 1), lambda i: (i, 0)),
                  pl.BlockSpec((2, 16), lambda i: (0, 0)),
                  pl.BlockSpec((16, c20), lambda i: (0, 0)),
                  pl.BlockSpec((tile, ns1, 16), lambda i: (i, 0, 0)),
                  pl.BlockSpec((tile, 16), lambda i: (i, 0)),
                  pl.BlockSpec((tile, 1), lambda i: (i, 0)),
                  pl.BlockSpec((2, 16), lambda i: (0, 0)),
                  pl.BlockSpec((16, c21), lambda i: (0, 0))],
        out_specs=[pl.BlockSpec((2, c20), lambda i: (0, 0)),
                   pl.BlockSpec((2, c21), lambda i: (0, 0)),
                   pl.BlockSpec((tile, c20), lambda i: (i, 0)),
                   pl.BlockSpec((tile, c21), lambda i: (i, 0))],
        out_shape=[jax.ShapeDtypeStruct((2, c20), jnp.float32),
                   jax.ShapeDtypeStruct((2, c21), jnp.float32),
                   jax.ShapeDtypeStruct((m, c20), jnp.float32),
                   jax.ShapeDtypeStruct((m, c21), jnp.float32)],
    )(g0, o0, m0, st0, w0t, g1, o1, m1, st1, w1t)


def _finalize(mx0, s20, mx1, s21, tile):
    m, c20 = mx0.shape
    c21 = mx1.shape[1]
    ns0, ns1 = _NSAMPLES
    n0, n1 = float(m * ns0), float(m * ns1)

    def body(mx0_ref, s20_ref, mx1_ref, s21_ref, out_ref):
        mu0, inv0 = _bn(s20_ref, n0)
        mu1, inv1 = _bn(s21_ref, n1)
        a = jnp.maximum((mx0_ref[...] - mu0) * inv0, 0.0)
        b = jnp.maximum((mx1_ref[...] - mu1) * inv1, 0.0)
        out_ref[...] = jnp.concatenate([a, b], axis=1)

    return pl.pallas_call(
        body,
        grid=(m // tile,),
        in_specs=[pl.BlockSpec((tile, c20), lambda i: (i, 0)),
                  pl.BlockSpec((2, c20), lambda i: (0, 0)),
                  pl.BlockSpec((tile, c21), lambda i: (i, 0)),
                  pl.BlockSpec((2, c21), lambda i: (0, 0))],
        out_specs=pl.BlockSpec((tile, c20 + c21), lambda i: (i, 0)),
        out_shape=jax.ShapeDtypeStruct((m, c20 + c21), jnp.float32),
    )(mx0, s20, mx1, s21)


def kernel(xyz, xyz_batch_cnt, new_xyz, new_xyz_batch_cnt, features,
           w0_0, w0_1, w1_0, w1_1):
    B = xyz_batch_cnt.shape[0]
    n_per = xyz.shape[0] // B
    m_per = new_xyz.shape[0] // B
    m = new_xyz.shape[0]

    u = jnp.concatenate([xyz, features], axis=1)
    h0, h1, o0, o1 = _precompute(u, new_xyz, w0_0.T, w1_0.T,
                                 w0_0[:, :3].T, w1_0[:, :3].T)

    g0f, g1f, cnt0, cnt1 = _sc_ball_gather(
        xyz[:, 0], xyz[:, 1], xyz[:, 2],
        new_xyz[:, 0], new_xyz[:, 1], new_xyz[:, 2],
        h0, h1, n_per, m_per)

    g0 = g0f.reshape(m, _NSAMPLES[0], 16)
    g1 = g1f.reshape(m, _NSAMPLES[1], 16)
    m0 = (cnt0 > 0).astype(jnp.float32).reshape(m, 1)
    m1 = (cnt1 > 0).astype(jnp.float32).reshape(m, 1)

    tile = 512
    st0, st1 = _stats1(g0, o0, m0, g1, o1, m1, tile)
    s20, s21, mx0, mx1 = _pass2(g0, o0, m0, st0, w0_1.T,
                                g1, o1, m1, st1, w1_1.T, tile)
    out = _finalize(mx0, s20, mx1, s21, 2048)

    return (new_xyz, out)


# speedup vs baseline: 33.8936x; 1.2299x over previous
import functools

import jax
import jax.numpy as jnp
from jax import lax
from jax.experimental import pallas as pl
from jax.experimental.pallas import tpu as pltpu
from jax.experimental.pallas import tpu_sc as plsc

_RADII = (0.8, 1.6)
_NSAMPLES = (16, 32)
_EPS = 1e-5

_NC = 2
_NSUB = 16
_NW = _NC * _NSUB
_L = 16
_RC = 128


def _splat(v, dtype=jnp.int32):
    return jnp.full((_L,), v, dtype=dtype)


def _precompute(u, new_xyz, w00t, w10t, wq0, wq1):
    n, ku = u.shape
    m, kq = new_xyz.shape
    grid = 16
    tn, tm = n // grid, m // grid

    def body(u_ref, q_ref, w00_ref, w10_ref, wq0_ref, wq1_ref,
             h0_ref, h1_ref, o0_ref, o1_ref):
        uu = u_ref[...]
        qq = q_ref[...]
        h0_ref[...] = jnp.dot(uu, w00_ref[...], preferred_element_type=jnp.float32)
        h1_ref[...] = jnp.dot(uu, w10_ref[...], preferred_element_type=jnp.float32)
        o0_ref[...] = jnp.dot(qq, wq0_ref[...], preferred_element_type=jnp.float32)
        o1_ref[...] = jnp.dot(qq, wq1_ref[...], preferred_element_type=jnp.float32)

    return pl.pallas_call(
        body,
        grid=(grid,),
        in_specs=[pl.BlockSpec((tn, ku), lambda i: (i, 0)),
                  pl.BlockSpec((tm, kq), lambda i: (i, 0)),
                  pl.BlockSpec((ku, 16), lambda i: (0, 0)),
                  pl.BlockSpec((ku, 16), lambda i: (0, 0)),
                  pl.BlockSpec((kq, 16), lambda i: (0, 0)),
                  pl.BlockSpec((kq, 16), lambda i: (0, 0))],
        out_specs=[pl.BlockSpec((tn, 16), lambda i: (i, 0)),
                   pl.BlockSpec((tn, 16), lambda i: (i, 0)),
                   pl.BlockSpec((tm, 16), lambda i: (i, 0)),
                   pl.BlockSpec((tm, 16), lambda i: (i, 0))],
        out_shape=[jax.ShapeDtypeStruct((n, 16), jnp.float32),
                   jax.ShapeDtypeStruct((n, 16), jnp.float32),
                   jax.ShapeDtypeStruct((m, 16), jnp.float32),
                   jax.ShapeDtypeStruct((m, 16), jnp.float32)],
    )(u, new_xyz, w00t, w10t, wq0, wq1)


def _sc_ball_gather(xs, ys, zs, qx, qy, qz, h0, h1, n_per, m_per):
    m = qx.shape[0]
    qw = m // _NW
    ns_max = _NSAMPLES[-1]
    mesh = plsc.VectorSubcoreMesh(core_axis_name="c", subcore_axis_name="s",
                                  num_cores=_NC, num_subcores=_NSUB)

    @functools.partial(
        pl.kernel,
        out_type=[
            jax.ShapeDtypeStruct((m * _NSAMPLES[0], 16), jnp.float32),
            jax.ShapeDtypeStruct((m * _NSAMPLES[1], 16), jnp.float32),
            jax.ShapeDtypeStruct((m,), jnp.int32),
            jax.ShapeDtypeStruct((m,), jnp.int32),
        ],
        mesh=mesh,
        compiler_params=pltpu.CompilerParams(needs_layout_passes=False,
                                             use_tc_tiling_on_sc=False),
        scratch_types=[
            pltpu.VMEM((n_per,), jnp.float32),
            pltpu.VMEM((n_per,), jnp.float32),
            pltpu.VMEM((n_per,), jnp.float32),
            pltpu.VMEM((qw,), jnp.float32),
            pltpu.VMEM((qw,), jnp.float32),
            pltpu.VMEM((qw,), jnp.float32),
            pltpu.VMEM((qw * ns_max,), jnp.int32),
            pltpu.VMEM((qw,), jnp.int32),
            pltpu.VMEM((_RC, 16), jnp.float32),
            pltpu.SemaphoreType.DMA,
        ],
    )
    def sc_kernel(xs_h, ys_h, zs_h, qx_h, qy_h, qz_h, h0_h, h1_h,
                  g0_h, g1_h, cnt0_h, cnt1_h,
                  xs_v, ys_v, zs_v, qx_v, qy_v, qz_v, idx_v, cnt_v, rows_v,
                  sem):
        wid = lax.axis_index("s") * _NC + lax.axis_index("c")
        qbase = wid * qw
        batch = qbase // m_per
        nbase = batch * n_per

        pltpu.sync_copy(xs_h.at[pl.ds(nbase, n_per)], xs_v)
        pltpu.sync_copy(ys_h.at[pl.ds(nbase, n_per)], ys_v)
        pltpu.sync_copy(zs_h.at[pl.ds(nbase, n_per)], zs_v)
        pltpu.sync_copy(qx_h.at[pl.ds(qbase, qw)], qx_v)
        pltpu.sync_copy(qy_h.at[pl.ds(qbase, qw)], qy_v)
        pltpu.sync_copy(qz_h.at[pl.ds(qbase, qw)], qz_v)

        lanes = lax.broadcasted_iota(jnp.int32, (_L,), 0)

        for scale, (radius, ns) in enumerate(zip(_RADII, _NSAMPLES)):
            r2 = radius * radius
            h_h = (h0_h, h1_h)[scale]
            g_h = (g0_h, g1_h)[scale]
            c_h = (cnt0_h, cnt1_h)[scale]

            def per_query(q, _, ns=ns, r2=r2):
                qi = _splat(q)
                qxv = plsc.load_gather(qx_v, [qi])
                qyv = plsc.load_gather(qy_v, [qi])
                qzv = plsc.load_gather(qz_v, [qi])

                def cond(carry):
                    j, cnt_s, _ = carry
                    return jnp.logical_and(cnt_s < ns, j < n_per)

                def scan_group(carry):
                    j, _, cntv = carry
                    for u in range(4):
                        jj = j + u * _L
                        dx = xs_v[pl.ds(jj, _L)] - qxv
                        dy = ys_v[pl.ds(jj, _L)] - qyv
                        dz = zs_v[pl.ds(jj, _L)] - qzv
                        d2 = dx * dx + dy * dy + dz * dz
                        within = d2 < r2
                        nhitv = plsc.all_reduce_population_count(within)
                        wi = within.astype(jnp.int32)
                        pos = plsc.cumsum(wi) - wi + cntv
                        smask = jnp.logical_and(within, pos < ns)
                        vals = lanes + (jj + nbase)
                        plsc.store_scatter(idx_v, [pos + q * ns], vals,
                                           mask=smask)
                        cntv = cntv + nhitv
                    return (j + 4 * _L, cntv[0], cntv)

                final_carry = lax.while_loop(
                    cond, scan_group,
                    (jnp.int32(0), jnp.int32(0), jnp.zeros((_L,), jnp.int32)))
                cnt = jnp.minimum(final_carry[1], ns)

                first = plsc.load_gather(idx_v, [_splat(q * ns)])
                firstv = jnp.where(_splat(cnt) > 0, first, _splat(nbase))
                cntv = _splat(cnt)
                for ch in range(ns // _L):
                    sl = pl.ds(q * ns + ch * _L, _L)
                    keep = (lanes + ch * _L) < cntv
                    idx_v[sl] = jnp.where(keep, idx_v[sl], firstv)
                plsc.store_scatter(cnt_v, [_splat(q)], cntv,
                                   mask=lanes == 0)
                return _

            lax.fori_loop(0, qw, per_query, 0)

            def per_chunk(c, _):
                cp = pltpu.async_copy(
                    h_h.at[idx_v.at[pl.ds(c * _RC, _RC)]], rows_v, sem)
                cp.wait()
                pltpu.sync_copy(rows_v,
                                g_h.at[pl.ds(qbase * ns + c * _RC, _RC)])
                return _

            lax.fori_loop(0, qw * ns // _RC, per_chunk, 0)
            pltpu.sync_copy(cnt_v, c_h.at[pl.ds(qbase, qw)])

    return sc_kernel(xs, ys, zs, qx, qy, qz, h0, h1)


def _y1(g_ref, o_ref, m_ref):
    return (g_ref[...] - o_ref[...][:, None, :]) * m_ref[...][:, :, None]


def _acc_sums(x, out_ref, axes):
    s = jnp.sum(x, axis=axes)
    sq = jnp.sum(x * x, axis=axes)
    blk = jnp.concatenate([s[None, :], sq[None, :]], axis=0)

    @pl.when(pl.program_id(0) == 0)
    def _init():
        out_ref[...] = jnp.zeros_like(out_ref)

    out_ref[...] += blk


def _stats1(g0, o0, m0, g1, o1, m1, tile):
    m = o0.shape[0]
    ns0, ns1 = _NSAMPLES

    def body(g0_ref, o0_ref, m0_ref, g1_ref, o1_ref, m1_ref,
             st0_ref, st1_ref):
        _acc_sums(_y1(g0_ref, o0_ref, m0_ref), st0_ref, (0, 1))
        _acc_sums(_y1(g1_ref, o1_ref, m1_ref), st1_ref, (0, 1))

    return pl.pallas_call(
        body,
        grid=(m // tile,),
        in_specs=[pl.BlockSpec((tile, ns0, 16), lambda i: (i, 0, 0)),
                  pl.BlockSpec((tile, 16), lambda i: (i, 0)),
                  pl.BlockSpec((tile, 1), lambda i: (i, 0)),
                  pl.BlockSpec((tile, ns1, 16), lambda i: (i, 0, 0)),
                  pl.BlockSpec((tile, 16), lambda i: (i, 0)),
                  pl.BlockSpec((tile, 1), lambda i: (i, 0))],
        out_specs=[pl.BlockSpec((2, 16), lambda i: (0, 0)),
                   pl.BlockSpec((2, 16), lambda i: (0, 0))],
        out_shape=[jax.ShapeDtypeStruct((2, 16), jnp.float32),
                   jax.ShapeDtypeStruct((2, 16), jnp.float32)],
    )(g0, o0, m0, g1, o1, m1)


def _bn(st_ref, n):
    mu = st_ref[0, :] / n
    var = st_ref[1, :] / n - mu * mu
    return mu, lax.rsqrt(var + _EPS)


def _pass2(g0, o0, m0, st0, w0t, g1, o1, m1, st1, w1t, tile):
    m = o0.shape[0]
    ns0, ns1 = _NSAMPLES
    n0, n1 = float(m * ns0), float(m * ns1)
    c20, c21 = w0t.shape[1], w1t.shape[1]

    def one(g_ref, o_ref, m_ref, st_ref, w_ref, s2_ref, mx_ref, ns, nn):
        mu1, inv1 = _bn(st_ref, nn)
        x1 = jnp.maximum((_y1(g_ref, o_ref, m_ref) - mu1) * inv1, 0.0)
        c2 = w_ref.shape[1]
        y2 = jnp.dot(x1.reshape(tile * ns, 16), w_ref[...],
                     preferred_element_type=jnp.float32)
        _acc_sums(y2, s2_ref, (0,))
        mx_ref[...] = jnp.max(y2.reshape(tile, ns, c2), axis=1)

    def body(g0_ref, o0_ref, m0_ref, st0_ref, w0_ref,
             g1_ref, o1_ref, m1_ref, st1_ref, w1_ref,
             s20_ref, s21_ref, mx0_ref, mx1_ref):
        one(g0_ref, o0_ref, m0_ref, st0_ref, w0_ref, s20_ref, mx0_ref,
            ns0, n0)
        one(g1_ref, o1_ref, m1_ref, st1_ref, w1_ref, s21_ref, mx1_ref,
            ns1, n1)

    return pl.pallas_call(
        body,
        grid=(m // tile,),
        in_specs=[pl.BlockSpec((tile, ns0, 16), lambda i: (i, 0, 0)),
                  pl.BlockSpec((tile, 16), lambda i: (i, 0)),
                  pl.BlockSpec((tile, 1), lambda i: (i, 0)),
                  pl.BlockSpec((2, 16), lambda i: (0, 0)),
                  pl.BlockSpec((16, c20), lambda i: (0, 0)),
                  pl.BlockSpec((tile, ns1, 16), lambda i: (i, 0, 0)),
                  pl.BlockSpec((tile, 16), lambda i: (i, 0)),
                  pl.BlockSpec((tile, 1), lambda i: (i, 0)),
                  pl.BlockSpec((2, 16), lambda i: (0, 0)),
                  pl.BlockSpec((16, c21), lambda i: (0, 0))],
        out_specs=[pl.BlockSpec((2, c20), lambda i: (0, 0)),
                   pl.BlockSpec((2, c21), lambda i: (0, 0)),
                   pl.BlockSpec((tile, c20), lambda i: (i, 0)),
                   pl.BlockSpec((tile, c21), lambda i: (i, 0))],
        out_shape=[jax.ShapeDtypeStruct((2, c20), jnp.float32),
                   jax.ShapeDtypeStruct((2, c21), jnp.float32),
                   jax.ShapeDtypeStruct((m, c20), jnp.float32),
                   jax.ShapeDtypeStruct((m, c21), jnp.float32)],
    )(g0, o0, m0, st0, w0t, g1, o1, m1, st1, w1t)


def _finalize(mx0, s20, mx1, s21, tile):
    m, c20 = mx0.shape
    c21 = mx1.shape[1]
    ns0, ns1 = _NSAMPLES
    n0, n1 = float(m * ns0), float(m * ns1)

    def body(mx0_ref, s20_ref, mx1_ref, s21_ref, out_ref):
        mu0, inv0 = _bn(s20_ref, n0)
        mu1, inv1 = _bn(s21_ref, n1)
        a = jnp.maximum((mx0_ref[...] - mu0) * inv0, 0.0)
        b = jnp.maximum((mx1_ref[...] - mu1) * inv1, 0.0)
        out_ref[...] = jnp.concatenate([a, b], axis=1)

    return pl.pallas_call(
        body,
        grid=(m // tile,),
        in_specs=[pl.BlockSpec((tile, c20), lambda i: (i, 0)),
                  pl.BlockSpec((2, c20), lambda i: (0, 0)),
                  pl.BlockSpec((tile, c21), lambda i: (i, 0)),
                  pl.BlockSpec((2, c21), lambda i: (0, 0))],
        out_specs=pl.BlockSpec((tile, c20 + c21), lambda i: (i, 0)),
        out_shape=jax.ShapeDtypeStruct((m, c20 + c21), jnp.float32),
    )(mx0, s20, mx1, s21)


def kernel(xyz, xyz_batch_cnt, new_xyz, new_xyz_batch_cnt, features,
           w0_0, w0_1, w1_0, w1_1):
    B = xyz_batch_cnt.shape[0]
    n_per = xyz.shape[0] // B
    m_per = new_xyz.shape[0] // B
    m = new_xyz.shape[0]

    u = jnp.concatenate([xyz, features], axis=1)
    h0, h1, o0, o1 = _precompute(u, new_xyz, w0_0.T, w1_0.T,
                                 w0_0[:, :3].T, w1_0[:, :3].T)

    g0f, g1f, cnt0, cnt1 = _sc_ball_gather(
        xyz[:, 0], xyz[:, 1], xyz[:, 2],
        new_xyz[:, 0], new_xyz[:, 1], new_xyz[:, 2],
        h0, h1, n_per, m_per)

    g0 = g0f.reshape(m, _NSAMPLES[0], 16)
    g1 = g1f.reshape(m, _NSAMPLES[1], 16)
    m0 = (cnt0 > 0).astype(jnp.float32).reshape(m, 1)
    m1 = (cnt1 > 0).astype(jnp.float32).reshape(m, 1)

    tile = 512
    st0, st1 = _stats1(g0, o0, m0, g1, o1, m1, tile)
    s20, s21, mx0, mx1 = _pass2(g0, o0, m0, st0, w0_1.T,
                                g1, o1, m1, st1, w1_1.T, tile)
    out = _finalize(mx0, s20, mx1, s21, 2048)

    return (new_xyz, out)


# speedup vs baseline: 42.9504x; 1.2672x over previous
import functools

import jax
import jax.numpy as jnp
from jax import lax
from jax.experimental import pallas as pl
from jax.experimental.pallas import tpu as pltpu
from jax.experimental.pallas import tpu_sc as plsc

_RADII = (0.8, 1.6)
_NSAMPLES = (16, 32)
_EPS = 1e-5

_NC = 2
_NSUB = 16
_NW = _NC * _NSUB
_L = 16
_RC = 128


def _splat(v, dtype=jnp.int32):
    return jnp.full((_L,), v, dtype=dtype)


def _precompute(u, new_xyz, w00t, w10t, wq0, wq1):
    n, ku = u.shape
    m, kq = new_xyz.shape
    grid = 16
    tn, tm = n // grid, m // grid

    def body(u_ref, q_ref, w00_ref, w10_ref, wq0_ref, wq1_ref,
             h0_ref, h1_ref, o0_ref, o1_ref):
        uu = u_ref[...]
        qq = q_ref[...]
        h0_ref[...] = jnp.dot(uu, w00_ref[...], preferred_element_type=jnp.float32)
        h1_ref[...] = jnp.dot(uu, w10_ref[...], preferred_element_type=jnp.float32)
        o0_ref[...] = jnp.dot(qq, wq0_ref[...], preferred_element_type=jnp.float32)
        o1_ref[...] = jnp.dot(qq, wq1_ref[...], preferred_element_type=jnp.float32)

    return pl.pallas_call(
        body,
        grid=(grid,),
        in_specs=[pl.BlockSpec((tn, ku), lambda i: (i, 0)),
                  pl.BlockSpec((tm, kq), lambda i: (i, 0)),
                  pl.BlockSpec((ku, 16), lambda i: (0, 0)),
                  pl.BlockSpec((ku, 16), lambda i: (0, 0)),
                  pl.BlockSpec((kq, 16), lambda i: (0, 0)),
                  pl.BlockSpec((kq, 16), lambda i: (0, 0))],
        out_specs=[pl.BlockSpec((tn, 16), lambda i: (i, 0)),
                   pl.BlockSpec((tn, 16), lambda i: (i, 0)),
                   pl.BlockSpec((tm, 16), lambda i: (i, 0)),
                   pl.BlockSpec((tm, 16), lambda i: (i, 0))],
        out_shape=[jax.ShapeDtypeStruct((n, 16), jnp.float32),
                   jax.ShapeDtypeStruct((n, 16), jnp.float32),
                   jax.ShapeDtypeStruct((m, 16), jnp.float32),
                   jax.ShapeDtypeStruct((m, 16), jnp.float32)],
    )(u, new_xyz, w00t, w10t, wq0, wq1)


def _sc_ball_gather(xs, ys, zs, qx, qy, qz, h0, h1, o0f, o1f, n_per, m_per):
    m = qx.shape[0]
    qw = m // _NW
    ns_max = _NSAMPLES[-1]
    mesh = plsc.VectorSubcoreMesh(core_axis_name="c", subcore_axis_name="s",
                                  num_cores=_NC, num_subcores=_NSUB)

    @functools.partial(
        pl.kernel,
        out_type=[
            jax.ShapeDtypeStruct((m * _NSAMPLES[0], 16), jnp.float32),
            jax.ShapeDtypeStruct((m * _NSAMPLES[1], 16), jnp.float32),
            jax.ShapeDtypeStruct((_NW * _L,), jnp.float32),
            jax.ShapeDtypeStruct((_NW * _L,), jnp.float32),
            jax.ShapeDtypeStruct((_NW * _L,), jnp.float32),
            jax.ShapeDtypeStruct((_NW * _L,), jnp.float32),
        ],
        mesh=mesh,
        compiler_params=pltpu.CompilerParams(needs_layout_passes=False,
                                             use_tc_tiling_on_sc=False),
        scratch_types=[
            pltpu.VMEM((n_per,), jnp.float32),
            pltpu.VMEM((n_per,), jnp.float32),
            pltpu.VMEM((n_per,), jnp.float32),
            pltpu.VMEM((qw,), jnp.float32),
            pltpu.VMEM((qw,), jnp.float32),
            pltpu.VMEM((qw,), jnp.float32),
            pltpu.VMEM((qw * _L,), jnp.float32),
            pltpu.VMEM((qw,), jnp.float32),
            pltpu.VMEM((qw * ns_max,), jnp.int32),
            pltpu.VMEM((_RC, 16), jnp.float32),
            pltpu.VMEM((_L,), jnp.float32),
            pltpu.VMEM((_L,), jnp.float32),
            pltpu.SemaphoreType.DMA,
        ],
    )
    def sc_kernel(xs_h, ys_h, zs_h, qx_h, qy_h, qz_h, h0_h, h1_h, o0_h, o1_h,
                  y0_h, y1_h, s10_h, sq0_h, s11_h, sq1_h,
                  xs_v, ys_v, zs_v, qx_v, qy_v, qz_v, o_v, mask_v, idx_v,
                  rows_v, s1_v, sq_v, sem):
        wid = lax.axis_index("s") * _NC + lax.axis_index("c")
        qbase = wid * qw
        batch = qbase // m_per
        nbase = batch * n_per

        pltpu.sync_copy(xs_h.at[pl.ds(nbase, n_per)], xs_v)
        pltpu.sync_copy(ys_h.at[pl.ds(nbase, n_per)], ys_v)
        pltpu.sync_copy(zs_h.at[pl.ds(nbase, n_per)], zs_v)
        pltpu.sync_copy(qx_h.at[pl.ds(qbase, qw)], qx_v)
        pltpu.sync_copy(qy_h.at[pl.ds(qbase, qw)], qy_v)
        pltpu.sync_copy(qz_h.at[pl.ds(qbase, qw)], qz_v)

        lanes = lax.broadcasted_iota(jnp.int32, (_L,), 0)
        onesf = jnp.full((_L,), 1.0, jnp.float32)
        zerosf = jnp.zeros((_L,), jnp.float32)

        for scale, (radius, ns) in enumerate(zip(_RADII, _NSAMPLES)):
            r2 = radius * radius
            h_h = (h0_h, h1_h)[scale]
            o_h = (o0_h, o1_h)[scale]
            y_h = (y0_h, y1_h)[scale]
            s_h = (s10_h, s11_h)[scale]
            q_h = (sq0_h, sq1_h)[scale]

            pltpu.sync_copy(o_h.at[pl.ds(qbase * _L, qw * _L)], o_v)

            def per_query(q, carry, ns=ns, r2=r2):
                qi = _splat(q)
                qxv = plsc.load_gather(qx_v, [qi])
                qyv = plsc.load_gather(qy_v, [qi])
                qzv = plsc.load_gather(qz_v, [qi])

                def cond(c):
                    j, cnt_s, _ = c
                    return jnp.logical_and(cnt_s < ns, j < n_per)

                def scan_group(c):
                    j, _, cntv = c
                    for u in range(4):
                        jj = j + u * _L
                        dx = xs_v[pl.ds(jj, _L)] - qxv
                        dy = ys_v[pl.ds(jj, _L)] - qyv
                        dz = zs_v[pl.ds(jj, _L)] - qzv
                        d2 = dx * dx + dy * dy + dz * dz
                        within = d2 < r2
                        nhitv = plsc.all_reduce_population_count(within)
                        wi = within.astype(jnp.int32)
                        pos = plsc.cumsum(wi) - wi + cntv
                        smask = jnp.logical_and(within, pos < ns)
                        vals = lanes + (jj + nbase)
                        plsc.store_scatter(idx_v, [pos + q * ns], vals,
                                           mask=smask)
                        cntv = cntv + nhitv
                    return (j + 4 * _L, cntv[0], cntv)

                fc = lax.while_loop(
                    cond, scan_group,
                    (jnp.int32(0), jnp.int32(0), jnp.zeros((_L,), jnp.int32)))
                cnt = jnp.minimum(fc[1], ns)

                first = plsc.load_gather(idx_v, [_splat(q * ns)])
                cntv = _splat(cnt)
                firstv = jnp.where(cntv > 0, first, _splat(nbase))
                for ch in range(ns // _L):
                    sl = pl.ds(q * ns + ch * _L, _L)
                    keep = (lanes + ch * _L) < cntv
                    idx_v[sl] = jnp.where(keep, idx_v[sl], firstv)
                plsc.store_scatter(mask_v, [_splat(q)],
                                   jnp.where(cntv > 0, onesf, zerosf),
                                   mask=lanes == 0)
                return carry

            lax.fori_loop(0, qw, per_query, 0)

            nsq = _RC // ns

            def per_chunk(c, carry, ns=ns, nsq=nsq):
                s1, sq = carry
                cp = pltpu.async_copy(
                    h_h.at[idx_v.at[pl.ds(c * _RC, _RC)]], rows_v, sem)
                cp.wait()
                for qq in range(nsq):
                    qloc = c * nsq + qq
                    ov = o_v[pl.ds(qloc * _L, _L)]
                    mk = plsc.load_gather(mask_v, [_splat(qloc)])
                    for s in range(ns):
                        row = qq * ns + s
                        y = (rows_v[row] - ov) * mk
                        rows_v[row] = y
                        s1 = s1 + y
                        sq = sq + y * y
                pltpu.sync_copy(rows_v,
                                y_h.at[pl.ds(qbase * ns + c * _RC, _RC)])
                return (s1, sq)

            s1, sq = lax.fori_loop(0, qw * ns // _RC, per_chunk,
                                   (zerosf, zerosf))
            s1_v[...] = s1
            sq_v[...] = sq
            pltpu.sync_copy(s1_v, s_h.at[pl.ds(wid * _L, _L)])
            pltpu.sync_copy(sq_v, q_h.at[pl.ds(wid * _L, _L)])

    return sc_kernel(xs, ys, zs, qx, qy, qz, h0, h1, o0f, o1f)


def _bn1_stats(s10, sq0, s11, sq1, n0, n1):
    def body(s10_ref, sq0_ref, s11_ref, sq1_ref, st0_ref, st1_ref):
        def one(s_ref, q_ref, out_ref, n):
            mu = jnp.sum(s_ref[...], axis=0) / n
            var = jnp.sum(q_ref[...], axis=0) / n - mu * mu
            inv = lax.rsqrt(var + _EPS)
            out_ref[...] = jnp.concatenate([mu[None, :], inv[None, :]], axis=0)
        one(s10_ref, sq0_ref, st0_ref, n0)
        one(s11_ref, sq1_ref, st1_ref, n1)

    spec = pl.BlockSpec((_NW, _L), lambda: (0, 0))
    ospec = pl.BlockSpec((2, _L), lambda: (0, 0))
    return pl.pallas_call(
        body,
        in_specs=[spec, spec, spec, spec],
        out_specs=[ospec, ospec],
        out_shape=[jax.ShapeDtypeStruct((2, _L), jnp.float32),
                   jax.ShapeDtypeStruct((2, _L), jnp.float32)],
    )(s10, sq0, s11, sq1)


def _pass2(y0p, mu0t, bd0, y1p, mu1t, bd1):
    r0 = y0p.shape[0]
    r1 = y1p.shape[0]
    c0 = bd0.shape[1]
    c1 = bd1.shape[1]
    ns0, ns1 = _NSAMPLES
    g0, g1 = ns0 // 8, ns1 // 8
    grid = 8
    t0r, t1r = r0 // grid, r1 // grid

    def body(y0_ref, m0_ref, b0_ref, y1_ref, m1_ref, b1_ref,
             s20_ref, s21_ref, mx0_ref, mx1_ref):
        def one(y_ref, m_ref, b_ref, s2_ref, mx_ref, gq, tr):
            t = jnp.maximum(y_ref[...] - m_ref[...], 0.0)
            y2 = jnp.dot(t, b_ref[...], preferred_element_type=jnp.float32)
            s = jnp.sum(y2, axis=0)
            q = jnp.sum(y2 * y2, axis=0)

            @pl.when(pl.program_id(0) == 0)
            def _init():
                s2_ref[...] = jnp.zeros_like(s2_ref)

            s2_ref[...] += jnp.concatenate([s[None, :], q[None, :]], axis=0)
            c = y2.shape[1]
            mx_ref[...] = jnp.max(y2.reshape(tr // gq, gq, c), axis=1)

        one(y0_ref, m0_ref, b0_ref, s20_ref, mx0_ref, g0, t0r)
        one(y1_ref, m1_ref, b1_ref, s21_ref, mx1_ref, g1, t1r)

    return pl.pallas_call(
        body,
        grid=(grid,),
        in_specs=[pl.BlockSpec((t0r, 128), lambda i: (i, 0)),
                  pl.BlockSpec((1, 128), lambda i: (0, 0)),
                  pl.BlockSpec((128, c0), lambda i: (0, 0)),
                  pl.BlockSpec((t1r, 128), lambda i: (i, 0)),
                  pl.BlockSpec((1, 128), lambda i: (0, 0)),
                  pl.BlockSpec((128, c1), lambda i: (0, 0))],
        out_specs=[pl.BlockSpec((2, c0), lambda i: (0, 0)),
                   pl.BlockSpec((2, c1), lambda i: (0, 0)),
                   pl.BlockSpec((t0r // g0, c0), lambda i: (i, 0)),
                   pl.BlockSpec((t1r // g1, c1), lambda i: (i, 0))],
        out_shape=[jax.ShapeDtypeStruct((2, c0), jnp.float32),
                   jax.ShapeDtypeStruct((2, c1), jnp.float32),
                   jax.ShapeDtypeStruct((r0 // g0, c0), jnp.float32),
                   jax.ShapeDtypeStruct((r1 // g1, c1), jnp.float32)],
    )(y0p, mu0t, bd0, y1p, mu1t, bd1)


def _finalize(mx0, s20, mx1, s21, tile):
    m = mx0.shape[0]
    ns0, ns1 = _NSAMPLES
    n0, n1 = float(m * ns0), float(m * ns1)

    def body(mx0_ref, s20_ref, mx1_ref, s21_ref, out_ref):
        def one(mx_ref, s2_ref, n, c2):
            st = jnp.sum(s2_ref[...].reshape(2, 8, c2), axis=1)
            mu = st[0, :] / n
            var = st[1, :] / n - mu * mu
            inv = lax.rsqrt(var + _EPS)
            mx = jnp.max(mx_ref[...].reshape(tile, 8, c2), axis=1)
            return jnp.maximum((mx - mu) * inv, 0.0)

        a = one(mx0_ref, s20_ref, n0, 16)
        b = one(mx1_ref, s21_ref, n1, 32)
        out_ref[...] = jnp.concatenate([a, b], axis=1)

    return pl.pallas_call(
        body,
        grid=(m // tile,),
        in_specs=[pl.BlockSpec((tile, 128), lambda i: (i, 0)),
                  pl.BlockSpec((2, 128), lambda i: (0, 0)),
                  pl.BlockSpec((tile, 256), lambda i: (i, 0)),
                  pl.BlockSpec((2, 256), lambda i: (0, 0))],
        out_specs=pl.BlockSpec((tile, 48), lambda i: (i, 0)),
        out_shape=jax.ShapeDtypeStruct((m, 48), jnp.float32),
    )(mx0, s20, mx1, s21)


def kernel(xyz, xyz_batch_cnt, new_xyz, new_xyz_batch_cnt, features,
           w0_0, w0_1, w1_0, w1_1):
    B = xyz_batch_cnt.shape[0]
    n_per = xyz.shape[0] // B
    m_per = new_xyz.shape[0] // B
    m = new_xyz.shape[0]
    ns0, ns1 = _NSAMPLES

    u = jnp.concatenate([xyz, features], axis=1)
    h0, h1, o0, o1 = _precompute(u, new_xyz, w0_0.T, w1_0.T,
                                 w0_0[:, :3].T, w1_0[:, :3].T)

    y0f, y1f, s10, sq0, s11, sq1 = _sc_ball_gather(
        xyz[:, 0], xyz[:, 1], xyz[:, 2],
        new_xyz[:, 0], new_xyz[:, 1], new_xyz[:, 2],
        h0, h1, o0.reshape(-1), o1.reshape(-1), n_per, m_per)

    st0, st1 = _bn1_stats(s10.reshape(_NW, _L), sq0.reshape(_NW, _L),
                          s11.reshape(_NW, _L), sq1.reshape(_NW, _L),
                          float(m * ns0), float(m * ns1))

    eye8 = jnp.eye(8, dtype=jnp.float32)
    bd0 = jnp.kron(eye8, st0[1][:, None] * w0_1.T)
    bd1 = jnp.kron(eye8, st1[1][:, None] * w1_1.T)
    mu0t = jnp.tile(st0[0], 8).reshape(1, 128)
    mu1t = jnp.tile(st1[0], 8).reshape(1, 128)

    s20, s21, mx0, mx1 = _pass2(y0f.reshape(m * ns0 // 8, 128), mu0t, bd0,
                                y1f.reshape(m * ns1 // 8, 128), mu1t, bd1)

    out = _finalize(mx0, s20, mx1, s21, 2048)
    return (new_xyz, out)


# speedup vs baseline: 62.7723x; 1.4615x over previous
import functools

import jax
import jax.numpy as jnp
from jax import lax
from jax.experimental import pallas as pl
from jax.experimental.pallas import tpu as pltpu
from jax.experimental.pallas import tpu_sc as plsc

_RADII = (0.8, 1.6)
_NSAMPLES = (16, 32)
_EPS = 1e-5

_NC = 2
_NSUB = 16
_NW = _NC * _NSUB
_L = 16
_RC = 128


def _splat(v, dtype=jnp.int32):
    return jnp.full((_L,), v, dtype=dtype)


def _precompute(u, new_xyz, w00t, w10t, wq0, wq1):
    n, ku = u.shape
    m, kq = new_xyz.shape
    grid = 16
    tn, tm = n // grid, m // grid

    def body(u_ref, q_ref, w00_ref, w10_ref, wq0_ref, wq1_ref,
             h0_ref, h1_ref, o0_ref, o1_ref):
        uu = u_ref[...]
        qq = q_ref[...]
        h0_ref[...] = jnp.dot(uu, w00_ref[...], preferred_element_type=jnp.float32)
        h1_ref[...] = jnp.dot(uu, w10_ref[...], preferred_element_type=jnp.float32)
        o0_ref[...] = jnp.dot(qq, wq0_ref[...], preferred_element_type=jnp.float32)
        o1_ref[...] = jnp.dot(qq, wq1_ref[...], preferred_element_type=jnp.float32)

    return pl.pallas_call(
        body,
        grid=(grid,),
        in_specs=[pl.BlockSpec((tn, ku), lambda i: (i, 0)),
                  pl.BlockSpec((tm, kq), lambda i: (i, 0)),
                  pl.BlockSpec((ku, 16), lambda i: (0, 0)),
                  pl.BlockSpec((ku, 16), lambda i: (0, 0)),
                  pl.BlockSpec((kq, 16), lambda i: (0, 0)),
                  pl.BlockSpec((kq, 16), lambda i: (0, 0))],
        out_specs=[pl.BlockSpec((tn, 16), lambda i: (i, 0)),
                   pl.BlockSpec((tn, 16), lambda i: (i, 0)),
                   pl.BlockSpec((tm, 16), lambda i: (i, 0)),
                   pl.BlockSpec((tm, 16), lambda i: (i, 0))],
        out_shape=[jax.ShapeDtypeStruct((n, 16), jnp.float32),
                   jax.ShapeDtypeStruct((n, 16), jnp.float32),
                   jax.ShapeDtypeStruct((m, 16), jnp.float32),
                   jax.ShapeDtypeStruct((m, 16), jnp.float32)],
    )(u, new_xyz, w00t, w10t, wq0, wq1)


def _sc_ball_gather(xs, ys, zs, qx, qy, qz, h0, h1, o0f, o1f, n_per, m_per):
    m = qx.shape[0]
    qw = m // _NW
    ns_max = _NSAMPLES[-1]
    mesh = plsc.VectorSubcoreMesh(core_axis_name="c", subcore_axis_name="s",
                                  num_cores=_NC, num_subcores=_NSUB)

    @functools.partial(
        pl.kernel,
        out_type=[
            jax.ShapeDtypeStruct((m * _NSAMPLES[0], 16), jnp.float32),
            jax.ShapeDtypeStruct((m * _NSAMPLES[1], 16), jnp.float32),
            jax.ShapeDtypeStruct((_NW * _L,), jnp.float32),
            jax.ShapeDtypeStruct((_NW * _L,), jnp.float32),
            jax.ShapeDtypeStruct((_NW * _L,), jnp.float32),
            jax.ShapeDtypeStruct((_NW * _L,), jnp.float32),
        ],
        mesh=mesh,
        compiler_params=pltpu.CompilerParams(needs_layout_passes=False,
                                             use_tc_tiling_on_sc=False),
        scratch_types=[
            pltpu.VMEM((n_per,), jnp.float32),
            pltpu.VMEM((n_per,), jnp.float32),
            pltpu.VMEM((n_per,), jnp.float32),
            pltpu.VMEM((qw,), jnp.float32),
            pltpu.VMEM((qw,), jnp.float32),
            pltpu.VMEM((qw,), jnp.float32),
            pltpu.VMEM((qw * _L,), jnp.float32),
            pltpu.VMEM((qw,), jnp.float32),
            pltpu.VMEM((qw * ns_max,), jnp.int32),
            pltpu.VMEM((_RC, 16), jnp.float32),
            pltpu.VMEM((_L,), jnp.float32),
            pltpu.VMEM((_L,), jnp.float32),
            pltpu.SemaphoreType.DMA,
        ],
    )
    def sc_kernel(xs_h, ys_h, zs_h, qx_h, qy_h, qz_h, h0_h, h1_h, o0_h, o1_h,
                  y0_h, y1_h, s10_h, sq0_h, s11_h, sq1_h,
                  xs_v, ys_v, zs_v, qx_v, qy_v, qz_v, o_v, mask_v, idx_v,
                  rows_v, s1_v, sq_v, sem):
        wid = lax.axis_index("s") * _NC + lax.axis_index("c")
        qbase = wid * qw
        batch = qbase // m_per
        nbase = batch * n_per

        pltpu.sync_copy(xs_h.at[pl.ds(nbase, n_per)], xs_v)
        pltpu.sync_copy(ys_h.at[pl.ds(nbase, n_per)], ys_v)
        pltpu.sync_copy(zs_h.at[pl.ds(nbase, n_per)], zs_v)
        pltpu.sync_copy(qx_h.at[pl.ds(qbase, qw)], qx_v)
        pltpu.sync_copy(qy_h.at[pl.ds(qbase, qw)], qy_v)
        pltpu.sync_copy(qz_h.at[pl.ds(qbase, qw)], qz_v)

        lanes = lax.broadcasted_iota(jnp.int32, (_L,), 0)
        onesf = jnp.full((_L,), 1.0, jnp.float32)
        zerosf = jnp.zeros((_L,), jnp.float32)

        for scale, (radius, ns) in enumerate(zip(_RADII, _NSAMPLES)):
            r2 = radius * radius
            h_h = (h0_h, h1_h)[scale]
            o_h = (o0_h, o1_h)[scale]
            y_h = (y0_h, y1_h)[scale]
            s_h = (s10_h, s11_h)[scale]
            q_h = (sq0_h, sq1_h)[scale]

            pltpu.sync_copy(o_h.at[pl.ds(qbase * _L, qw * _L)], o_v)

            def per_query(q, carry, ns=ns, r2=r2):
                qi = _splat(q)
                qxv = plsc.load_gather(qx_v, [qi])
                qyv = plsc.load_gather(qy_v, [qi])
                qzv = plsc.load_gather(qz_v, [qi])

                def cond(c):
                    j, cnt_s, _ = c
                    return jnp.logical_and(cnt_s < ns, j < n_per)

                ones_i = jnp.full((_L,), 1, jnp.int32)
                row_ref = idx_v.at[pl.ds(q * ns, ns)]

                def scan_group(c):
                    j, cnt_s, cntv = c
                    masks, bases = [], []
                    for u in range(4):
                        jj = j + u * _L
                        dx = xs_v[pl.ds(jj, _L)] - qxv
                        dy = ys_v[pl.ds(jj, _L)] - qyv
                        dz = zs_v[pl.ds(jj, _L)] - qzv
                        d2 = dx * dx + dy * dy + dz * dz
                        within = d2 < r2
                        masks.append(within)
                        bases.append(cntv)
                        cntv = cntv + plsc.all_reduce_population_count(within)
                    cnt_s2 = cntv[0]

                    @pl.when(cnt_s2 > cnt_s)
                    def _emit():
                        for u in range(4):
                            pos = plsc.cumsum(ones_i, mask=masks[u]) - 1 \
                                + bases[u]
                            smask = jnp.logical_and(masks[u], pos < ns)
                            vals = lanes + ((j + u * _L) + nbase)
                            plsc.store_scatter(row_ref, [pos], vals,
                                               mask=smask)

                    return (j + 4 * _L, cnt_s2, cntv)

                fc = lax.while_loop(
                    cond, scan_group,
                    (jnp.int32(0), jnp.int32(0), jnp.zeros((_L,), jnp.int32)))
                cnt = jnp.minimum(fc[1], ns)

                first = plsc.load_gather(idx_v, [_splat(q * ns)])
                cntv = _splat(cnt)
                firstv = jnp.where(cntv > 0, first, _splat(nbase))
                for ch in range(ns // _L):
                    sl = pl.ds(q * ns + ch * _L, _L)
                    keep = (lanes + ch * _L) < cntv
                    idx_v[sl] = jnp.where(keep, idx_v[sl], firstv)
                plsc.store_scatter(mask_v, [_splat(q)],
                                   jnp.where(cntv > 0, onesf, zerosf),
                                   mask=lanes == 0)
                return carry

            lax.fori_loop(0, qw, per_query, 0)

            nsq = _RC // ns

            def per_chunk(c, carry, ns=ns, nsq=nsq):
                s1, sq = carry
                cp = pltpu.async_copy(
                    h_h.at[idx_v.at[pl.ds(c * _RC, _RC)]], rows_v, sem)
                cp.wait()
                for qq in range(nsq):
                    qloc = c * nsq + qq
                    ov = o_v[pl.ds(qloc * _L, _L)]
                    mk = plsc.load_gather(mask_v, [_splat(qloc)])
                    for s in range(ns):
                        row = qq * ns + s
                        y = (rows_v[row] - ov) * mk
                        rows_v[row] = y
                        s1 = s1 + y
                        sq = sq + y * y
                pltpu.sync_copy(rows_v,
                                y_h.at[pl.ds(qbase * ns + c * _RC, _RC)])
                return (s1, sq)

            s1, sq = lax.fori_loop(0, qw * ns // _RC, per_chunk,
                                   (zerosf, zerosf))
            s1_v[...] = s1
            sq_v[...] = sq
            pltpu.sync_copy(s1_v, s_h.at[pl.ds(wid * _L, _L)])
            pltpu.sync_copy(sq_v, q_h.at[pl.ds(wid * _L, _L)])

    return sc_kernel(xs, ys, zs, qx, qy, qz, h0, h1, o0f, o1f)


def _bn1_stats(s10, sq0, s11, sq1, n0, n1):
    def body(s10_ref, sq0_ref, s11_ref, sq1_ref, st0_ref, st1_ref):
        def one(s_ref, q_ref, out_ref, n):
            mu = jnp.sum(s_ref[...], axis=0) / n
            var = jnp.sum(q_ref[...], axis=0) / n - mu * mu
            inv = lax.rsqrt(var + _EPS)
            out_ref[...] = jnp.concatenate([mu[None, :], inv[None, :]], axis=0)
        one(s10_ref, sq0_ref, st0_ref, n0)
        one(s11_ref, sq1_ref, st1_ref, n1)

    spec = pl.BlockSpec((_NW, _L), lambda: (0, 0))
    ospec = pl.BlockSpec((2, _L), lambda: (0, 0))
    return pl.pallas_call(
        body,
        in_specs=[spec, spec, spec, spec],
        out_specs=[ospec, ospec],
        out_shape=[jax.ShapeDtypeStruct((2, _L), jnp.float32),
                   jax.ShapeDtypeStruct((2, _L), jnp.float32)],
    )(s10, sq0, s11, sq1)


def _pass2(y0p, mu0t, bd0, y1p, mu1t, bd1):
    r0 = y0p.shape[0]
    r1 = y1p.shape[0]
    c0 = bd0.shape[1]
    c1 = bd1.shape[1]
    ns0, ns1 = _NSAMPLES
    g0, g1 = ns0 // 8, ns1 // 8
    grid = 8
    t0r, t1r = r0 // grid, r1 // grid

    def body(y0_ref, m0_ref, b0_ref, y1_ref, m1_ref, b1_ref,
             s20_ref, s21_ref, mx0_ref, mx1_ref):
        def one(y_ref, m_ref, b_ref, s2_ref, mx_ref, gq, tr):
            t = jnp.maximum(y_ref[...] - m_ref[...], 0.0)
            y2 = jnp.dot(t, b_ref[...], preferred_element_type=jnp.float32)
            s = jnp.sum(y2, axis=0)
            q = jnp.sum(y2 * y2, axis=0)

            @pl.when(pl.program_id(0) == 0)
            def _init():
                s2_ref[...] = jnp.zeros_like(s2_ref)

            s2_ref[...] += jnp.concatenate([s[None, :], q[None, :]], axis=0)
            c = y2.shape[1]
            mx_ref[...] = jnp.max(y2.reshape(tr // gq, gq, c), axis=1)

        one(y0_ref, m0_ref, b0_ref, s20_ref, mx0_ref, g0, t0r)
        one(y1_ref, m1_ref, b1_ref, s21_ref, mx1_ref, g1, t1r)

    return pl.pallas_call(
        body,
        grid=(grid,),
        in_specs=[pl.BlockSpec((t0r, 128), lambda i: (i, 0)),
                  pl.BlockSpec((1, 128), lambda i: (0, 0)),
                  pl.BlockSpec((128, c0), lambda i: (0, 0)),
                  pl.BlockSpec((t1r, 128), lambda i: (i, 0)),
                  pl.BlockSpec((1, 128), lambda i: (0, 0)),
                  pl.BlockSpec((128, c1), lambda i: (0, 0))],
        out_specs=[pl.BlockSpec((2, c0), lambda i: (0, 0)),
                   pl.BlockSpec((2, c1), lambda i: (0, 0)),
                   pl.BlockSpec((t0r // g0, c0), lambda i: (i, 0)),
                   pl.BlockSpec((t1r // g1, c1), lambda i: (i, 0))],
        out_shape=[jax.ShapeDtypeStruct((2, c0), jnp.float32),
                   jax.ShapeDtypeStruct((2, c1), jnp.float32),
                   jax.ShapeDtypeStruct((r0 // g0, c0), jnp.float32),
                   jax.ShapeDtypeStruct((r1 // g1, c1), jnp.float32)],
    )(y0p, mu0t, bd0, y1p, mu1t, bd1)


def _finalize(mx0, s20, mx1, s21, tile):
    m = mx0.shape[0]
    ns0, ns1 = _NSAMPLES
    n0, n1 = float(m * ns0), float(m * ns1)

    def body(mx0_ref, s20_ref, mx1_ref, s21_ref, out_ref):
        def one(mx_ref, s2_ref, n, c2):
            st = jnp.sum(s2_ref[...].reshape(2, 8, c2), axis=1)
            mu = st[0, :] / n
            var = st[1, :] / n - mu * mu
            inv = lax.rsqrt(var + _EPS)
            mx = jnp.max(mx_ref[...].reshape(tile, 8, c2), axis=1)
            return jnp.maximum((mx - mu) * inv, 0.0)

        a = one(mx0_ref, s20_ref, n0, 16)
        b = one(mx1_ref, s21_ref, n1, 32)
        out_ref[...] = jnp.concatenate([a, b], axis=1)

    return pl.pallas_call(
        body,
        grid=(m // tile,),
        in_specs=[pl.BlockSpec((tile, 128), lambda i: (i, 0)),
                  pl.BlockSpec((2, 128), lambda i: (0, 0)),
                  pl.BlockSpec((tile, 256), lambda i: (i, 0)),
                  pl.BlockSpec((2, 256), lambda i: (0, 0))],
        out_specs=pl.BlockSpec((tile, 48), lambda i: (i, 0)),
        out_shape=jax.ShapeDtypeStruct((m, 48), jnp.float32),
    )(mx0, s20, mx1, s21)


def kernel(xyz, xyz_batch_cnt, new_xyz, new_xyz_batch_cnt, features,
           w0_0, w0_1, w1_0, w1_1):
    B = xyz_batch_cnt.shape[0]
    n_per = xyz.shape[0] // B
    m_per = new_xyz.shape[0] // B
    m = new_xyz.shape[0]
    ns0, ns1 = _NSAMPLES

    u = jnp.concatenate([xyz, features], axis=1)
    h0, h1, o0, o1 = _precompute(u, new_xyz, w0_0.T, w1_0.T,
                                 w0_0[:, :3].T, w1_0[:, :3].T)

    y0f, y1f, s10, sq0, s11, sq1 = _sc_ball_gather(
        xyz[:, 0], xyz[:, 1], xyz[:, 2],
        new_xyz[:, 0], new_xyz[:, 1], new_xyz[:, 2],
        h0, h1, o0.reshape(-1), o1.reshape(-1), n_per, m_per)

    st0, st1 = _bn1_stats(s10.reshape(_NW, _L), sq0.reshape(_NW, _L),
                          s11.reshape(_NW, _L), sq1.reshape(_NW, _L),
                          float(m * ns0), float(m * ns1))

    eye8 = jnp.eye(8, dtype=jnp.float32)
    bd0 = jnp.kron(eye8, st0[1][:, None] * w0_1.T)
    bd1 = jnp.kron(eye8, st1[1][:, None] * w1_1.T)
    mu0t = jnp.tile(st0[0], 8).reshape(1, 128)
    mu1t = jnp.tile(st1[0], 8).reshape(1, 128)

    s20, s21, mx0, mx1 = _pass2(y0f.reshape(m * ns0 // 8, 128), mu0t, bd0,
                                y1f.reshape(m * ns1 // 8, 128), mu1t, bd1)

    out = _finalize(mx0, s20, mx1, s21, 2048)
    return (new_xyz, out)


# speedup vs baseline: 69.4629x; 1.1066x over previous
import functools

import jax
import jax.numpy as jnp
from jax import lax
from jax.experimental import pallas as pl
from jax.experimental.pallas import tpu as pltpu
from jax.experimental.pallas import tpu_sc as plsc

_RADII = (0.8, 1.6)
_NSAMPLES = (16, 32)
_EPS = 1e-5

_NC = 2
_NSUB = 16
_NW = _NC * _NSUB
_L = 16
_RC = 128


def _splat(v, dtype=jnp.int32):
    return jnp.full((_L,), v, dtype=dtype)


def _precompute(u, new_xyz, w00t, w10t, wq0, wq1):
    n, ku = u.shape
    m, kq = new_xyz.shape
    grid = 16
    tn, tm = n // grid, m // grid

    def body(u_ref, q_ref, w00_ref, w10_ref, wq0_ref, wq1_ref,
             h0_ref, h1_ref, o0_ref, o1_ref):
        uu = u_ref[...]
        qq = q_ref[...]
        h0_ref[...] = jnp.dot(uu, w00_ref[...], preferred_element_type=jnp.float32)
        h1_ref[...] = jnp.dot(uu, w10_ref[...], preferred_element_type=jnp.float32)
        o0_ref[...] = jnp.dot(qq, wq0_ref[...], preferred_element_type=jnp.float32)
        o1_ref[...] = jnp.dot(qq, wq1_ref[...], preferred_element_type=jnp.float32)

    return pl.pallas_call(
        body,
        grid=(grid,),
        in_specs=[pl.BlockSpec((tn, ku), lambda i: (i, 0)),
                  pl.BlockSpec((tm, kq), lambda i: (i, 0)),
                  pl.BlockSpec((ku, 16), lambda i: (0, 0)),
                  pl.BlockSpec((ku, 16), lambda i: (0, 0)),
                  pl.BlockSpec((kq, 16), lambda i: (0, 0)),
                  pl.BlockSpec((kq, 16), lambda i: (0, 0))],
        out_specs=[pl.BlockSpec((tn, 16), lambda i: (i, 0)),
                   pl.BlockSpec((tn, 16), lambda i: (i, 0)),
                   pl.BlockSpec((tm, 16), lambda i: (i, 0)),
                   pl.BlockSpec((tm, 16), lambda i: (i, 0))],
        out_shape=[jax.ShapeDtypeStruct((n, 16), jnp.float32),
                   jax.ShapeDtypeStruct((n, 16), jnp.float32),
                   jax.ShapeDtypeStruct((m, 16), jnp.float32),
                   jax.ShapeDtypeStruct((m, 16), jnp.float32)],
    )(u, new_xyz, w00t, w10t, wq0, wq1)


def _sc_ball_gather(xs, ys, zs, qx, qy, qz, h0, h1, o0f, o1f, n_per, m_per):
    m = qx.shape[0]
    qw = m // _NW
    ns_max = _NSAMPLES[-1]
    mesh = plsc.VectorSubcoreMesh(core_axis_name="c", subcore_axis_name="s",
                                  num_cores=_NC, num_subcores=_NSUB)

    @functools.partial(
        pl.kernel,
        out_type=[
            jax.ShapeDtypeStruct((m * _NSAMPLES[0], 16), jnp.float32),
            jax.ShapeDtypeStruct((m * _NSAMPLES[1], 16), jnp.float32),
            jax.ShapeDtypeStruct((_NW * _L,), jnp.float32),
            jax.ShapeDtypeStruct((_NW * _L,), jnp.float32),
            jax.ShapeDtypeStruct((_NW * _L,), jnp.float32),
            jax.ShapeDtypeStruct((_NW * _L,), jnp.float32),
        ],
        mesh=mesh,
        compiler_params=pltpu.CompilerParams(needs_layout_passes=False,
                                             use_tc_tiling_on_sc=False),
        scratch_types=[
            pltpu.VMEM((n_per,), jnp.float32),
            pltpu.VMEM((n_per,), jnp.float32),
            pltpu.VMEM((n_per,), jnp.float32),
            pltpu.VMEM((qw,), jnp.float32),
            pltpu.VMEM((qw,), jnp.float32),
            pltpu.VMEM((qw,), jnp.float32),
            pltpu.VMEM((qw * _L,), jnp.float32),
            pltpu.VMEM((qw,), jnp.float32),
            pltpu.VMEM((qw * ns_max,), jnp.int32),
            pltpu.VMEM((_RC, 16), jnp.float32),
            pltpu.VMEM((_RC, 16), jnp.float32),
            pltpu.VMEM((_L,), jnp.float32),
            pltpu.VMEM((_L,), jnp.float32),
            pltpu.SemaphoreType.DMA,
            pltpu.SemaphoreType.DMA,
        ],
    )
    def sc_kernel(xs_h, ys_h, zs_h, qx_h, qy_h, qz_h, h0_h, h1_h, o0_h, o1_h,
                  y0_h, y1_h, s10_h, sq0_h, s11_h, sq1_h,
                  xs_v, ys_v, zs_v, qx_v, qy_v, qz_v, o_v, mask_v, idx_v,
                  rows_v, rows2_v, s1_v, sq_v, sem_a, sem_b):
        wid = lax.axis_index("s") * _NC + lax.axis_index("c")
        qbase = wid * qw
        batch = qbase // m_per
        nbase = batch * n_per

        stage = [
            pltpu.async_copy(xs_h.at[pl.ds(nbase, n_per)], xs_v, sem_a),
            pltpu.async_copy(ys_h.at[pl.ds(nbase, n_per)], ys_v, sem_a),
            pltpu.async_copy(zs_h.at[pl.ds(nbase, n_per)], zs_v, sem_a),
            pltpu.async_copy(qx_h.at[pl.ds(qbase, qw)], qx_v, sem_a),
            pltpu.async_copy(qy_h.at[pl.ds(qbase, qw)], qy_v, sem_a),
            pltpu.async_copy(qz_h.at[pl.ds(qbase, qw)], qz_v, sem_a),
        ]
        for cp in stage:
            cp.wait()

        lanes = lax.broadcasted_iota(jnp.int32, (_L,), 0)
        onesf = jnp.full((_L,), 1.0, jnp.float32)
        zerosf = jnp.zeros((_L,), jnp.float32)

        for scale, (radius, ns) in enumerate(zip(_RADII, _NSAMPLES)):
            r2 = radius * radius
            h_h = (h0_h, h1_h)[scale]
            o_h = (o0_h, o1_h)[scale]
            y_h = (y0_h, y1_h)[scale]
            s_h = (s10_h, s11_h)[scale]
            q_h = (sq0_h, sq1_h)[scale]

            pltpu.sync_copy(o_h.at[pl.ds(qbase * _L, qw * _L)], o_v)

            def per_query(q, carry, ns=ns, r2=r2):
                qi = _splat(q)
                qxv = plsc.load_gather(qx_v, [qi])
                qyv = plsc.load_gather(qy_v, [qi])
                qzv = plsc.load_gather(qz_v, [qi])

                def cond(c):
                    j, cnt_s, _ = c
                    return jnp.logical_and(cnt_s < ns, j < n_per)

                ones_i = jnp.full((_L,), 1, jnp.int32)
                row_ref = idx_v.at[pl.ds(q * ns, ns)]

                def scan_group(c):
                    j, cnt_s, cntv = c
                    masks, bases = [], []
                    for u in range(4):
                        jj = j + u * _L
                        dx = xs_v[pl.ds(jj, _L)] - qxv
                        dy = ys_v[pl.ds(jj, _L)] - qyv
                        dz = zs_v[pl.ds(jj, _L)] - qzv
                        d2 = dx * dx + dy * dy + dz * dz
                        within = d2 < r2
                        masks.append(within)
                        bases.append(cntv)
                        cntv = cntv + plsc.all_reduce_population_count(within)
                    cnt_s2 = cntv[0]

                    @pl.when(cnt_s2 > cnt_s)
                    def _emit():
                        for u in range(4):
                            pos = plsc.cumsum(ones_i, mask=masks[u]) - 1 \
                                + bases[u]
                            smask = jnp.logical_and(masks[u], pos < ns)
                            vals = lanes + ((j + u * _L) + nbase)
                            plsc.store_scatter(row_ref, [pos], vals,
                                               mask=smask)

                    return (j + 4 * _L, cnt_s2, cntv)

                fc = lax.while_loop(
                    cond, scan_group,
                    (jnp.int32(0), jnp.int32(0), jnp.zeros((_L,), jnp.int32)))
                cnt = jnp.minimum(fc[1], ns)

                first = plsc.load_gather(idx_v, [_splat(q * ns)])
                cntv = _splat(cnt)
                firstv = jnp.where(cntv > 0, first, _splat(nbase))
                for ch in range(ns // _L):
                    sl = pl.ds(q * ns + ch * _L, _L)
                    keep = (lanes + ch * _L) < cntv
                    idx_v[sl] = jnp.where(keep, idx_v[sl], firstv)
                plsc.store_scatter(mask_v, [_splat(q)],
                                   jnp.where(cntv > 0, onesf, zerosf),
                                   mask=lanes == 0)
                return carry

            lax.fori_loop(0, qw, per_query, 0)

            nsq = _RC // ns
            nch = qw * ns // _RC
            bufs = (rows_v, rows2_v)
            sems = (sem_a, sem_b)

            def _issue(c, b):
                pltpu.async_copy(h_h.at[idx_v.at[pl.ds(c * _RC, _RC)]],
                                 bufs[b], sems[b])

            _issue(0, 0)

            def per_pair(p, carry, ns=ns, nsq=nsq, nch=nch):
                s1, sq = carry
                for b in range(2):
                    c = 2 * p + b
                    buf = bufs[b]

                    @pl.when(c + 1 < nch)
                    def _prefetch():
                        _issue(c + 1, 1 - b)

                    pltpu.make_async_copy(
                        h_h.at[idx_v.at[pl.ds(c * _RC, _RC)]], buf,
                        sems[b]).wait()
                    for qq in range(nsq):
                        qloc = c * nsq + qq
                        ov = o_v[pl.ds(qloc * _L, _L)]
                        mk = plsc.load_gather(mask_v, [_splat(qloc)])
                        for s in range(ns):
                            row = qq * ns + s
                            y = (buf[row] - ov) * mk
                            buf[row] = y
                            s1 = s1 + y
                            sq = sq + y * y
                    pltpu.sync_copy(buf,
                                    y_h.at[pl.ds(qbase * ns + c * _RC, _RC)])
                return (s1, sq)

            s1, sq = lax.fori_loop(0, nch // 2, per_pair,
                                   (zerosf, zerosf))
            s1_v[...] = s1
            sq_v[...] = sq
            pltpu.sync_copy(s1_v, s_h.at[pl.ds(wid * _L, _L)])
            pltpu.sync_copy(sq_v, q_h.at[pl.ds(wid * _L, _L)])

    return sc_kernel(xs, ys, zs, qx, qy, qz, h0, h1, o0f, o1f)


def _bn1_stats(s10, sq0, s11, sq1, n0, n1):
    def body(s10_ref, sq0_ref, s11_ref, sq1_ref, st0_ref, st1_ref):
        def one(s_ref, q_ref, out_ref, n):
            mu = jnp.sum(s_ref[...], axis=0) / n
            var = jnp.sum(q_ref[...], axis=0) / n - mu * mu
            inv = lax.rsqrt(var + _EPS)
            out_ref[...] = jnp.concatenate([mu[None, :], inv[None, :]], axis=0)
        one(s10_ref, sq0_ref, st0_ref, n0)
        one(s11_ref, sq1_ref, st1_ref, n1)

    spec = pl.BlockSpec((_NW, _L), lambda: (0, 0))
    ospec = pl.BlockSpec((2, _L), lambda: (0, 0))
    return pl.pallas_call(
        body,
        in_specs=[spec, spec, spec, spec],
        out_specs=[ospec, ospec],
        out_shape=[jax.ShapeDtypeStruct((2, _L), jnp.float32),
                   jax.ShapeDtypeStruct((2, _L), jnp.float32)],
    )(s10, sq0, s11, sq1)


def _pass2(y0p, mu0t, bd0, y1p, mu1t, bd1):
    r0 = y0p.shape[0]
    r1 = y1p.shape[0]
    c0 = bd0.shape[1]
    c1 = bd1.shape[1]
    ns0, ns1 = _NSAMPLES
    g0, g1 = ns0 // 8, ns1 // 8
    grid = 8
    t0r, t1r = r0 // grid, r1 // grid

    def body(y0_ref, m0_ref, b0_ref, y1_ref, m1_ref, b1_ref,
             s20_ref, s21_ref, mx0_ref, mx1_ref):
        def one(y_ref, m_ref, b_ref, s2_ref, mx_ref, gq, tr):
            t = jnp.maximum(y_ref[...] - m_ref[...], 0.0)
            y2 = jnp.dot(t, b_ref[...], preferred_element_type=jnp.float32)
            s = jnp.sum(y2, axis=0)
            q = jnp.sum(y2 * y2, axis=0)

            @pl.when(pl.program_id(0) == 0)
            def _init():
                s2_ref[...] = jnp.zeros_like(s2_ref)

            s2_ref[...] += jnp.concatenate([s[None, :], q[None, :]], axis=0)
            c = y2.shape[1]
            mx_ref[...] = jnp.max(y2.reshape(tr // gq, gq, c), axis=1)

        one(y0_ref, m0_ref, b0_ref, s20_ref, mx0_ref, g0, t0r)
        one(y1_ref, m1_ref, b1_ref, s21_ref, mx1_ref, g1, t1r)

    return pl.pallas_call(
        body,
        grid=(grid,),
        in_specs=[pl.BlockSpec((t0r, 128), lambda i: (i, 0)),
                  pl.BlockSpec((1, 128), lambda i: (0, 0)),
                  pl.BlockSpec((128, c0), lambda i: (0, 0)),
                  pl.BlockSpec((t1r, 128), lambda i: (i, 0)),
                  pl.BlockSpec((1, 128), lambda i: (0, 0)),
                  pl.BlockSpec((128, c1), lambda i: (0, 0))],
        out_specs=[pl.BlockSpec((2, c0), lambda i: (0, 0)),
                   pl.BlockSpec((2, c1), lambda i: (0, 0)),
                   pl.BlockSpec((t0r // g0, c0), lambda i: (i, 0)),
                   pl.BlockSpec((t1r // g1, c1), lambda i: (i, 0))],
        out_shape=[jax.ShapeDtypeStruct((2, c0), jnp.float32),
                   jax.ShapeDtypeStruct((2, c1), jnp.float32),
                   jax.ShapeDtypeStruct((r0 // g0, c0), jnp.float32),
                   jax.ShapeDtypeStruct((r1 // g1, c1), jnp.float32)],
    )(y0p, mu0t, bd0, y1p, mu1t, bd1)


def _finalize(mx0, s20, mx1, s21, tile):
    m = mx0.shape[0]
    ns0, ns1 = _NSAMPLES
    n0, n1 = float(m * ns0), float(m * ns1)

    def body(mx0_ref, s20_ref, mx1_ref, s21_ref, out_ref):
        def one(mx_ref, s2_ref, n, c2):
            st = jnp.sum(s2_ref[...].reshape(2, 8, c2), axis=1)
            mu = st[0, :] / n
            var = st[1, :] / n - mu * mu
            inv = lax.rsqrt(var + _EPS)
            mx = jnp.max(mx_ref[...].reshape(tile, 8, c2), axis=1)
            return jnp.maximum((mx - mu) * inv, 0.0)

        a = one(mx0_ref, s20_ref, n0, 16)
        b = one(mx1_ref, s21_ref, n1, 32)
        out_ref[...] = jnp.concatenate([a, b], axis=1)

    return pl.pallas_call(
        body,
        grid=(m // tile,),
        in_specs=[pl.BlockSpec((tile, 128), lambda i: (i, 0)),
                  pl.BlockSpec((2, 128), lambda i: (0, 0)),
                  pl.BlockSpec((tile, 256), lambda i: (i, 0)),
                  pl.BlockSpec((2, 256), lambda i: (0, 0))],
        out_specs=pl.BlockSpec((tile, 48), lambda i: (i, 0)),
        out_shape=jax.ShapeDtypeStruct((m, 48), jnp.float32),
    )(mx0, s20, mx1, s21)


def kernel(xyz, xyz_batch_cnt, new_xyz, new_xyz_batch_cnt, features,
           w0_0, w0_1, w1_0, w1_1):
    B = xyz_batch_cnt.shape[0]
    n_per = xyz.shape[0] // B
    m_per = new_xyz.shape[0] // B
    m = new_xyz.shape[0]
    ns0, ns1 = _NSAMPLES

    u = jnp.concatenate([xyz, features], axis=1)
    h0, h1, o0, o1 = _precompute(u, new_xyz, w0_0.T, w1_0.T,
                                 w0_0[:, :3].T, w1_0[:, :3].T)

    y0f, y1f, s10, sq0, s11, sq1 = _sc_ball_gather(
        xyz[:, 0], xyz[:, 1], xyz[:, 2],
        new_xyz[:, 0], new_xyz[:, 1], new_xyz[:, 2],
        h0, h1, o0.reshape(-1), o1.reshape(-1), n_per, m_per)

    st0, st1 = _bn1_stats(s10.reshape(_NW, _L), sq0.reshape(_NW, _L),
                          s11.reshape(_NW, _L), sq1.reshape(_NW, _L),
                          float(m * ns0), float(m * ns1))

    eye8 = jnp.eye(8, dtype=jnp.float32)
    bd0 = jnp.kron(eye8, st0[1][:, None] * w0_1.T)
    bd1 = jnp.kron(eye8, st1[1][:, None] * w1_1.T)
    mu0t = jnp.tile(st0[0], 8).reshape(1, 128)
    mu1t = jnp.tile(st1[0], 8).reshape(1, 128)

    s20, s21, mx0, mx1 = _pass2(y0f.reshape(m * ns0 // 8, 128), mu0t, bd0,
                                y1f.reshape(m * ns1 // 8, 128), mu1t, bd1)

    out = _finalize(mx0, s20, mx1, s21, 2048)
    return (new_xyz, out)


# speedup vs baseline: 70.0073x; 1.0078x over previous
import functools

import jax
import jax.numpy as jnp
from jax import lax
from jax.experimental import pallas as pl
from jax.experimental.pallas import tpu as pltpu
from jax.experimental.pallas import tpu_sc as plsc

_RADII = (0.8, 1.6)
_NSAMPLES = (16, 32)
_EPS = 1e-5

_NC = 2
_NSUB = 16
_NW = _NC * _NSUB
_L = 16
_RC = 128


def _splat(v, dtype=jnp.int32):
    return jnp.full((_L,), v, dtype=dtype)


def _precompute(u, new_xyz, w00t, w10t, wq0, wq1):
    n, ku = u.shape
    m, kq = new_xyz.shape
    grid = 16
    tn, tm = n // grid, m // grid

    def body(u_ref, q_ref, w00_ref, w10_ref, wq0_ref, wq1_ref,
             h0_ref, h1_ref, o0_ref, o1_ref):
        uu = u_ref[...]
        qq = q_ref[...]
        h0_ref[...] = jnp.dot(uu, w00_ref[...], preferred_element_type=jnp.float32)
        h1_ref[...] = jnp.dot(uu, w10_ref[...], preferred_element_type=jnp.float32)
        o0_ref[...] = jnp.dot(qq, wq0_ref[...], preferred_element_type=jnp.float32)
        o1_ref[...] = jnp.dot(qq, wq1_ref[...], preferred_element_type=jnp.float32)

    return pl.pallas_call(
        body,
        grid=(grid,),
        in_specs=[pl.BlockSpec((tn, ku), lambda i: (i, 0)),
                  pl.BlockSpec((tm, kq), lambda i: (i, 0)),
                  pl.BlockSpec((ku, 16), lambda i: (0, 0)),
                  pl.BlockSpec((ku, 16), lambda i: (0, 0)),
                  pl.BlockSpec((kq, 16), lambda i: (0, 0)),
                  pl.BlockSpec((kq, 16), lambda i: (0, 0))],
        out_specs=[pl.BlockSpec((tn, 16), lambda i: (i, 0)),
                   pl.BlockSpec((tn, 16), lambda i: (i, 0)),
                   pl.BlockSpec((tm, 16), lambda i: (i, 0)),
                   pl.BlockSpec((tm, 16), lambda i: (i, 0))],
        out_shape=[jax.ShapeDtypeStruct((n, 16), jnp.float32),
                   jax.ShapeDtypeStruct((n, 16), jnp.float32),
                   jax.ShapeDtypeStruct((m, 16), jnp.float32),
                   jax.ShapeDtypeStruct((m, 16), jnp.float32)],
    )(u, new_xyz, w00t, w10t, wq0, wq1)


def _sc_ball_gather(xs, ys, zs, qx, qy, qz, h0, h1, o0f, o1f, n_per, m_per):
    m = qx.shape[0]
    qw = m // _NW
    ns_max = _NSAMPLES[-1]
    mesh = plsc.VectorSubcoreMesh(core_axis_name="c", subcore_axis_name="s",
                                  num_cores=_NC, num_subcores=_NSUB)

    @functools.partial(
        pl.kernel,
        out_type=[
            jax.ShapeDtypeStruct((m * _NSAMPLES[0], 16), jnp.float32),
            jax.ShapeDtypeStruct((m * _NSAMPLES[1], 16), jnp.float32),
            jax.ShapeDtypeStruct((_NW * _L,), jnp.float32),
            jax.ShapeDtypeStruct((_NW * _L,), jnp.float32),
            jax.ShapeDtypeStruct((_NW * _L,), jnp.float32),
            jax.ShapeDtypeStruct((_NW * _L,), jnp.float32),
        ],
        mesh=mesh,
        compiler_params=pltpu.CompilerParams(needs_layout_passes=False,
                                             use_tc_tiling_on_sc=False),
        scratch_types=[
            pltpu.VMEM((n_per,), jnp.float32),
            pltpu.VMEM((n_per,), jnp.float32),
            pltpu.VMEM((n_per,), jnp.float32),
            pltpu.VMEM((qw,), jnp.float32),
            pltpu.VMEM((qw,), jnp.float32),
            pltpu.VMEM((qw,), jnp.float32),
            pltpu.VMEM((qw * _L,), jnp.float32),
            pltpu.VMEM((qw,), jnp.float32),
            pltpu.VMEM((qw * ns_max,), jnp.int32),
            pltpu.VMEM((_RC, 16), jnp.float32),
            pltpu.VMEM((_RC, 16), jnp.float32),
            pltpu.VMEM((_L,), jnp.float32),
            pltpu.VMEM((_L,), jnp.float32),
            pltpu.SemaphoreType.DMA,
            pltpu.SemaphoreType.DMA,
        ],
    )
    def sc_kernel(xs_h, ys_h, zs_h, qx_h, qy_h, qz_h, h0_h, h1_h, o0_h, o1_h,
                  y0_h, y1_h, s10_h, sq0_h, s11_h, sq1_h,
                  xs_v, ys_v, zs_v, qx_v, qy_v, qz_v, o_v, mask_v, idx_v,
                  rows_v, rows2_v, s1_v, sq_v, sem_a, sem_b):
        wid = lax.axis_index("s") * _NC + lax.axis_index("c")
        qbase = wid * qw
        batch = qbase // m_per
        nbase = batch * n_per

        stage = [
            pltpu.async_copy(xs_h.at[pl.ds(nbase, n_per)], xs_v, sem_a),
            pltpu.async_copy(ys_h.at[pl.ds(nbase, n_per)], ys_v, sem_a),
            pltpu.async_copy(zs_h.at[pl.ds(nbase, n_per)], zs_v, sem_a),
            pltpu.async_copy(qx_h.at[pl.ds(qbase, qw)], qx_v, sem_a),
            pltpu.async_copy(qy_h.at[pl.ds(qbase, qw)], qy_v, sem_a),
            pltpu.async_copy(qz_h.at[pl.ds(qbase, qw)], qz_v, sem_a),
        ]
        for cp in stage:
            cp.wait()

        lanes = lax.broadcasted_iota(jnp.int32, (_L,), 0)
        onesf = jnp.full((_L,), 1.0, jnp.float32)
        zerosf = jnp.zeros((_L,), jnp.float32)

        for scale, (radius, ns) in enumerate(zip(_RADII, _NSAMPLES)):
            r2 = radius * radius
            h_h = (h0_h, h1_h)[scale]
            o_h = (o0_h, o1_h)[scale]
            y_h = (y0_h, y1_h)[scale]
            s_h = (s10_h, s11_h)[scale]
            q_h = (sq0_h, sq1_h)[scale]

            pltpu.sync_copy(o_h.at[pl.ds(qbase * _L, qw * _L)], o_v)

            def per_query(q, carry, ns=ns, r2=r2):
                qi = _splat(q)
                qxv = plsc.load_gather(qx_v, [qi])
                qyv = plsc.load_gather(qy_v, [qi])
                qzv = plsc.load_gather(qz_v, [qi])

                def cond(c):
                    j, cnt_s, _ = c
                    return jnp.logical_and(cnt_s < ns, j < n_per)

                ones_i = jnp.full((_L,), 1, jnp.int32)
                row_ref = idx_v.at[pl.ds(q * ns, ns)]

                def scan_group(c):
                    j, cnt_s, cntv = c
                    masks, bases = [], []
                    for u in range(4):
                        jj = j + u * _L
                        dx = xs_v[pl.ds(jj, _L)] - qxv
                        dy = ys_v[pl.ds(jj, _L)] - qyv
                        dz = zs_v[pl.ds(jj, _L)] - qzv
                        d2 = dx * dx + dy * dy + dz * dz
                        within = d2 < r2
                        masks.append(within)
                        bases.append(cntv)
                        cntv = cntv + plsc.all_reduce_population_count(within)
                    cnt_s2 = cntv[0]

                    @pl.when(cnt_s2 > cnt_s)
                    def _emit():
                        for u in range(4):
                            pos = plsc.cumsum(ones_i, mask=masks[u]) - 1 \
                                + bases[u]
                            smask = jnp.logical_and(masks[u], pos < ns)
                            vals = lanes + ((j + u * _L) + nbase)
                            plsc.store_scatter(row_ref, [pos], vals,
                                               mask=smask)

                    return (j + 4 * _L, cnt_s2, cntv)

                fc = lax.while_loop(
                    cond, scan_group,
                    (jnp.int32(0), jnp.int32(0), jnp.zeros((_L,), jnp.int32)))
                cnt = jnp.minimum(fc[1], ns)

                first = plsc.load_gather(idx_v, [_splat(q * ns)])
                cntv = _splat(cnt)
                firstv = jnp.where(cntv > 0, first, _splat(nbase))
                for ch in range(ns // _L):
                    sl = pl.ds(q * ns + ch * _L, _L)
                    keep = (lanes + ch * _L) < cntv
                    idx_v[sl] = jnp.where(keep, idx_v[sl], firstv)
                plsc.store_scatter(mask_v, [_splat(q)],
                                   jnp.where(cntv > 0, onesf, zerosf),
                                   mask=lanes == 0)
                return carry

            lax.fori_loop(0, qw, per_query, 0)

            nsq = _RC // ns
            nch = qw * ns // _RC
            bufs = (rows_v, rows2_v)
            sems = (sem_a, sem_b)

            def _issue(c, b):
                pltpu.async_copy(h_h.at[idx_v.at[pl.ds(c * _RC, _RC)]],
                                 bufs[b], sems[b])

            _issue(0, 0)

            def per_pair(p, carry, ns=ns, nsq=nsq, nch=nch):
                s1, sq = carry
                for b in range(2):
                    c = 2 * p + b
                    buf = bufs[b]

                    @pl.when(c + 1 < nch)
                    def _prefetch():
                        _issue(c + 1, 1 - b)

                    pltpu.make_async_copy(
                        h_h.at[idx_v.at[pl.ds(c * _RC, _RC)]], buf,
                        sems[b]).wait()
                    for qq in range(nsq):
                        qloc = c * nsq + qq
                        ov = o_v[pl.ds(qloc * _L, _L)]
                        mk = plsc.load_gather(mask_v, [_splat(qloc)])
                        for s in range(ns):
                            row = qq * ns + s
                            y = (buf[row] - ov) * mk
                            buf[row] = y
                            s1 = s1 + y
                            sq = sq + y * y
                    pltpu.sync_copy(buf,
                                    y_h.at[pl.ds(qbase * ns + c * _RC, _RC)])
                return (s1, sq)

            s1, sq = lax.fori_loop(0, nch // 2, per_pair,
                                   (zerosf, zerosf))
            s1_v[...] = s1
            sq_v[...] = sq
            pltpu.sync_copy(s1_v, s_h.at[pl.ds(wid * _L, _L)])
            pltpu.sync_copy(sq_v, q_h.at[pl.ds(wid * _L, _L)])

    return sc_kernel(xs, ys, zs, qx, qy, qz, h0, h1, o0f, o1f)


def _mlp_fused(y0p, y1p, s10, sq0, s11, sq1, k0, k1, m):
    r0 = y0p.shape[0]
    r1 = y1p.shape[0]
    c0 = k0.shape[1]
    c1 = k1.shape[1]
    ns0, ns1 = _NSAMPLES
    n0, n1 = float(m * ns0), float(m * ns1)
    g0, g1 = ns0 // 8, ns1 // 8
    grid = 8
    t0r, t1r = r0 // grid, r1 // grid
    mq = m // grid

    def body(y0_ref, y1_ref, s10_ref, sq0_ref, s11_ref, sq1_ref,
             k0_ref, k1_ref, s20_ref, s21_ref, mx0_ref, mx1_ref):
        i = pl.program_id(0)

        def bn1(s_ref, q_ref, n):
            mu = jnp.sum(s_ref[...], axis=0) / n
            var = jnp.sum(q_ref[...], axis=0) / n - mu * mu
            return mu, lax.rsqrt(var + _EPS)

        def one(y_ref, s_ref, q_ref, k_ref, s2_ref, mx_ref, n, gq, tr):
            mu, inv = bn1(s_ref, q_ref, n)
            mut = jnp.concatenate([mu] * 8).reshape(1, 128)
            invc = jnp.concatenate([inv] * 8).reshape(128, 1)
            t = jnp.maximum(y_ref[...] - mut, 0.0)
            y2 = jnp.dot(t, k_ref[...] * invc,
                         preferred_element_type=jnp.float32)
            s = jnp.sum(y2, axis=0)
            q = jnp.sum(y2 * y2, axis=0)

            @pl.when(i == 0)
            def _init():
                s2_ref[...] = jnp.zeros_like(s2_ref)

            s2_ref[...] += jnp.concatenate([s[None, :], q[None, :]], axis=0)
            c = y2.shape[1]
            mx_ref[...] = jnp.max(y2.reshape(tr // gq, gq, c), axis=1)

        one(y0_ref, s10_ref, sq0_ref, k0_ref, s20_ref, mx0_ref, n0, g0, t0r)
        one(y1_ref, s11_ref, sq1_ref, k1_ref, s21_ref, mx1_ref, n1, g1, t1r)

    sspec = pl.BlockSpec((_NW, _L), lambda i: (0, 0))
    return pl.pallas_call(
        body,
        grid=(grid,),
        in_specs=[pl.BlockSpec((t0r, 128), lambda i: (i, 0)),
                  pl.BlockSpec((t1r, 128), lambda i: (i, 0)),
                  sspec, sspec, sspec, sspec,
                  pl.BlockSpec((128, c0), lambda i: (0, 0)),
                  pl.BlockSpec((128, c1), lambda i: (0, 0))],
        out_specs=[pl.BlockSpec((2, c0), lambda i: (0, 0)),
                   pl.BlockSpec((2, c1), lambda i: (0, 0)),
                   pl.BlockSpec((mq, c0), lambda i: (i, 0)),
                   pl.BlockSpec((mq, c1), lambda i: (i, 0))],
        out_shape=[jax.ShapeDtypeStruct((2, c0), jnp.float32),
                   jax.ShapeDtypeStruct((2, c1), jnp.float32),
                   jax.ShapeDtypeStruct((m, c0), jnp.float32),
                   jax.ShapeDtypeStruct((m, c1), jnp.float32)],
    )(y0p, y1p, s10, sq0, s11, sq1, k0, k1)


def _finalize(mx0, s20, mx1, s21, tile):
    m = mx0.shape[0]
    ns0, ns1 = _NSAMPLES
    n0, n1 = float(m * ns0), float(m * ns1)

    def body(mx0_ref, s20_ref, mx1_ref, s21_ref, out_ref):
        def one(mx_ref, s2_ref, n, c2):
            st = jnp.sum(s2_ref[...].reshape(2, 8, c2), axis=1)
            mu = st[0, :] / n
            var = st[1, :] / n - mu * mu
            inv = lax.rsqrt(var + _EPS)
            mx = jnp.max(mx_ref[...].reshape(tile, 8, c2), axis=1)
            return jnp.maximum((mx - mu) * inv, 0.0)

        a = one(mx0_ref, s20_ref, n0, 16)
        b = one(mx1_ref, s21_ref, n1, 32)
        out_ref[...] = jnp.concatenate([a, b], axis=1)

    return pl.pallas_call(
        body,
        grid=(m // tile,),
        in_specs=[pl.BlockSpec((tile, 128), lambda i: (i, 0)),
                  pl.BlockSpec((2, 128), lambda i: (0, 0)),
                  pl.BlockSpec((tile, 256), lambda i: (i, 0)),
                  pl.BlockSpec((2, 256), lambda i: (0, 0))],
        out_specs=pl.BlockSpec((tile, 48), lambda i: (i, 0)),
        out_shape=jax.ShapeDtypeStruct((m, 48), jnp.float32),
    )(mx0, s20, mx1, s21)


def kernel(xyz, xyz_batch_cnt, new_xyz, new_xyz_batch_cnt, features,
           w0_0, w0_1, w1_0, w1_1):
    B = xyz_batch_cnt.shape[0]
    n_per = xyz.shape[0] // B
    m_per = new_xyz.shape[0] // B
    m = new_xyz.shape[0]
    ns0, ns1 = _NSAMPLES

    u = jnp.concatenate([xyz, features], axis=1)
    h0, h1, o0, o1 = _precompute(u, new_xyz, w0_0.T, w1_0.T,
                                 w0_0[:, :3].T, w1_0[:, :3].T)

    y0f, y1f, s10, sq0, s11, sq1 = _sc_ball_gather(
        xyz[:, 0], xyz[:, 1], xyz[:, 2],
        new_xyz[:, 0], new_xyz[:, 1], new_xyz[:, 2],
        h0, h1, o0.reshape(-1), o1.reshape(-1), n_per, m_per)

    eye8 = jnp.eye(8, dtype=jnp.float32)
    k0 = jnp.kron(eye8, w0_1.T)
    k1 = jnp.kron(eye8, w1_1.T)

    s20, s21, mx0, mx1 = _mlp_fused(
        y0f.reshape(m * ns0 // 8, 128), y1f.reshape(m * ns1 // 8, 128),
        s10.reshape(_NW, _L), sq0.reshape(_NW, _L),
        s11.reshape(_NW, _L), sq1.reshape(_NW, _L), k0, k1, m)
    out = _finalize(mx0, s20, mx1, s21, 2048)
    return (new_xyz, out)


# speedup vs baseline: 82.2043x; 1.1742x over previous
import functools

import jax
import jax.numpy as jnp
from jax import lax
from jax.experimental import pallas as pl
from jax.experimental.pallas import tpu as pltpu
from jax.experimental.pallas import tpu_sc as plsc

_RADII = (0.8, 1.6)
_NSAMPLES = (16, 32)
_EPS = 1e-5

_NC = 2
_NSUB = 16
_NW = _NC * _NSUB
_L = 16
_RC = 128
_GU = 8


def _splat(v, dtype=jnp.int32):
    return jnp.full((_L,), v, dtype=dtype)


def _precompute(u, new_xyz, w00t, w10t, wq0, wq1):
    n, ku = u.shape
    m, kq = new_xyz.shape
    grid = 16
    tn, tm = n // grid, m // grid

    def body(u_ref, q_ref, w00_ref, w10_ref, wq0_ref, wq1_ref,
             h0_ref, h1_ref, o0_ref, o1_ref):
        uu = u_ref[...]
        qq = q_ref[...]
        h0_ref[...] = jnp.dot(uu, w00_ref[...], preferred_element_type=jnp.float32)
        h1_ref[...] = jnp.dot(uu, w10_ref[...], preferred_element_type=jnp.float32)
        o0_ref[...] = jnp.dot(qq, wq0_ref[...], preferred_element_type=jnp.float32)
        o1_ref[...] = jnp.dot(qq, wq1_ref[...], preferred_element_type=jnp.float32)

    return pl.pallas_call(
        body,
        grid=(grid,),
        in_specs=[pl.BlockSpec((tn, ku), lambda i: (i, 0)),
                  pl.BlockSpec((tm, kq), lambda i: (i, 0)),
                  pl.BlockSpec((ku, 16), lambda i: (0, 0)),
                  pl.BlockSpec((ku, 16), lambda i: (0, 0)),
                  pl.BlockSpec((kq, 16), lambda i: (0, 0)),
                  pl.BlockSpec((kq, 16), lambda i: (0, 0))],
        out_specs=[pl.BlockSpec((tn, 16), lambda i: (i, 0)),
                   pl.BlockSpec((tn, 16), lambda i: (i, 0)),
                   pl.BlockSpec((tm, 16), lambda i: (i, 0)),
                   pl.BlockSpec((tm, 16), lambda i: (i, 0))],
        out_shape=[jax.ShapeDtypeStruct((n, 16), jnp.float32),
                   jax.ShapeDtypeStruct((n, 16), jnp.float32),
                   jax.ShapeDtypeStruct((m, 16), jnp.float32),
                   jax.ShapeDtypeStruct((m, 16), jnp.float32)],
    )(u, new_xyz, w00t, w10t, wq0, wq1)


def _sc_ball_gather(xs, ys, zs, qx, qy, qz, h0, h1, o0f, o1f, n_per, m_per):
    m = qx.shape[0]
    qw = m // _NW
    ns_max = _NSAMPLES[-1]
    mesh = plsc.VectorSubcoreMesh(core_axis_name="c", subcore_axis_name="s",
                                  num_cores=_NC, num_subcores=_NSUB)

    @functools.partial(
        pl.kernel,
        out_type=[
            jax.ShapeDtypeStruct((m * _NSAMPLES[0], 16), jnp.float32),
            jax.ShapeDtypeStruct((m * _NSAMPLES[1], 16), jnp.float32),
            jax.ShapeDtypeStruct((_NW * _L,), jnp.float32),
            jax.ShapeDtypeStruct((_NW * _L,), jnp.float32),
            jax.ShapeDtypeStruct((_NW * _L,), jnp.float32),
            jax.ShapeDtypeStruct((_NW * _L,), jnp.float32),
        ],
        mesh=mesh,
        compiler_params=pltpu.CompilerParams(needs_layout_passes=False,
                                             use_tc_tiling_on_sc=False),
        scratch_types=[
            pltpu.VMEM((n_per,), jnp.float32),
            pltpu.VMEM((n_per,), jnp.float32),
            pltpu.VMEM((n_per,), jnp.float32),
            pltpu.VMEM((qw,), jnp.float32),
            pltpu.VMEM((qw,), jnp.float32),
            pltpu.VMEM((qw,), jnp.float32),
            pltpu.VMEM((qw * _L,), jnp.float32),
            pltpu.VMEM((qw,), jnp.float32),
            pltpu.VMEM((qw * ns_max,), jnp.int32),
            pltpu.VMEM((_RC, 16), jnp.float32),
            pltpu.VMEM((_RC, 16), jnp.float32),
            pltpu.VMEM((_L,), jnp.float32),
            pltpu.VMEM((_L,), jnp.float32),
            pltpu.SemaphoreType.DMA,
            pltpu.SemaphoreType.DMA,
        ],
    )
    def sc_kernel(xs_h, ys_h, zs_h, qx_h, qy_h, qz_h, h0_h, h1_h, o0_h, o1_h,
                  y0_h, y1_h, s10_h, sq0_h, s11_h, sq1_h,
                  xs_v, ys_v, zs_v, qx_v, qy_v, qz_v, o_v, mask_v, idx_v,
                  rows_v, rows2_v, s1_v, sq_v, sem_a, sem_b):
        wid = lax.axis_index("s") * _NC + lax.axis_index("c")
        qbase = wid * qw
        batch = qbase // m_per
        nbase = batch * n_per

        stage = [
            pltpu.async_copy(xs_h.at[pl.ds(nbase, n_per)], xs_v, sem_a),
            pltpu.async_copy(ys_h.at[pl.ds(nbase, n_per)], ys_v, sem_a),
            pltpu.async_copy(zs_h.at[pl.ds(nbase, n_per)], zs_v, sem_a),
            pltpu.async_copy(qx_h.at[pl.ds(qbase, qw)], qx_v, sem_a),
            pltpu.async_copy(qy_h.at[pl.ds(qbase, qw)], qy_v, sem_a),
            pltpu.async_copy(qz_h.at[pl.ds(qbase, qw)], qz_v, sem_a),
        ]
        for cp in stage:
            cp.wait()

        lanes = lax.broadcasted_iota(jnp.int32, (_L,), 0)
        onesf = jnp.full((_L,), 1.0, jnp.float32)
        zerosf = jnp.zeros((_L,), jnp.float32)

        for scale, (radius, ns) in enumerate(zip(_RADII, _NSAMPLES)):
            r2 = radius * radius
            h_h = (h0_h, h1_h)[scale]
            o_h = (o0_h, o1_h)[scale]
            y_h = (y0_h, y1_h)[scale]
            s_h = (s10_h, s11_h)[scale]
            q_h = (sq0_h, sq1_h)[scale]

            pltpu.sync_copy(o_h.at[pl.ds(qbase * _L, qw * _L)], o_v)

            def per_query(q, carry, ns=ns, r2=r2):
                qi = _splat(q)
                qxv = plsc.load_gather(qx_v, [qi])
                qyv = plsc.load_gather(qy_v, [qi])
                qzv = plsc.load_gather(qz_v, [qi])

                def cond(c):
                    j, cnt_s, _ = c
                    return jnp.logical_and(cnt_s < ns, j < n_per)

                ones_i = jnp.full((_L,), 1, jnp.int32)
                row_ref = idx_v.at[pl.ds(q * ns, ns)]

                def scan_group(c):
                    j, cnt_s, cntv = c
                    masks, bases = [], []
                    for u in range(_GU):
                        jj = j + u * _L
                        dx = xs_v[pl.ds(jj, _L)] - qxv
                        dy = ys_v[pl.ds(jj, _L)] - qyv
                        dz = zs_v[pl.ds(jj, _L)] - qzv
                        d2 = dx * dx + dy * dy + dz * dz
                        within = d2 < r2
                        masks.append(within)
                        bases.append(cntv)
                        cntv = cntv + plsc.all_reduce_population_count(within)
                    cnt_s2 = cntv[0]

                    @pl.when(cnt_s2 > cnt_s)
                    def _emit():
                        for u in range(_GU):
                            pos = plsc.cumsum(ones_i, mask=masks[u]) - 1 \
                                + bases[u]
                            smask = jnp.logical_and(masks[u], pos < ns)
                            vals = lanes + ((j + u * _L) + nbase)
                            plsc.store_scatter(row_ref, [pos], vals,
                                               mask=smask)

                    return (j + _GU * _L, cnt_s2, cntv)

                fc = lax.while_loop(
                    cond, scan_group,
                    (jnp.int32(0), jnp.int32(0), jnp.zeros((_L,), jnp.int32)))
                cnt = jnp.minimum(fc[1], ns)

                first = plsc.load_gather(idx_v, [_splat(q * ns)])
                cntv = _splat(cnt)
                firstv = jnp.where(cntv > 0, first, _splat(nbase))
                for ch in range(ns // _L):
                    sl = pl.ds(q * ns + ch * _L, _L)
                    keep = (lanes + ch * _L) < cntv
                    idx_v[sl] = jnp.where(keep, idx_v[sl], firstv)
                plsc.store_scatter(mask_v, [_splat(q)],
                                   jnp.where(cntv > 0, onesf, zerosf),
                                   mask=lanes == 0)
                return carry

            lax.fori_loop(0, qw, per_query, 0)

            nsq = _RC // ns
            nch = qw * ns // _RC
            bufs = (rows_v, rows2_v)
            sems = (sem_a, sem_b)

            def _issue(c, b):
                pltpu.async_copy(h_h.at[idx_v.at[pl.ds(c * _RC, _RC)]],
                                 bufs[b], sems[b])

            _issue(0, 0)

            def per_pair(p, carry, ns=ns, nsq=nsq, nch=nch):
                s1, sq = carry
                for b in range(2):
                    c = 2 * p + b
                    buf = bufs[b]

                    @pl.when(c + 1 < nch)
                    def _prefetch():
                        _issue(c + 1, 1 - b)

                    pltpu.make_async_copy(
                        h_h.at[idx_v.at[pl.ds(c * _RC, _RC)]], buf,
                        sems[b]).wait()
                    for qq in range(nsq):
                        qloc = c * nsq + qq
                        ov = o_v[pl.ds(qloc * _L, _L)]
                        mk = plsc.load_gather(mask_v, [_splat(qloc)])
                        for s in range(ns):
                            row = qq * ns + s
                            y = (buf[row] - ov) * mk
                            buf[row] = y
                            s1 = s1 + y
                            sq = sq + y * y
                    pltpu.sync_copy(buf,
                                    y_h.at[pl.ds(qbase * ns + c * _RC, _RC)])
                return (s1, sq)

            s1, sq = lax.fori_loop(0, nch // 2, per_pair,
                                   (zerosf, zerosf))
            s1_v[...] = s1
            sq_v[...] = sq
            pltpu.sync_copy(s1_v, s_h.at[pl.ds(wid * _L, _L)])
            pltpu.sync_copy(sq_v, q_h.at[pl.ds(wid * _L, _L)])

    return sc_kernel(xs, ys, zs, qx, qy, qz, h0, h1, o0f, o1f)


def _mlp_fused(y0p, y1p, s10, sq0, s11, sq1, k0, k1, m):
    r0 = y0p.shape[0]
    r1 = y1p.shape[0]
    c0 = k0.shape[1]
    c1 = k1.shape[1]
    ns0, ns1 = _NSAMPLES
    n0, n1 = float(m * ns0), float(m * ns1)
    g0, g1 = ns0 // 8, ns1 // 8
    grid = 8
    t0r, t1r = r0 // grid, r1 // grid
    mq = m // grid

    def body(y0_ref, y1_ref, s10_ref, sq0_ref, s11_ref, sq1_ref,
             k0_ref, k1_ref, s20_ref, s21_ref, mx0_ref, mx1_ref):
        i = pl.program_id(0)

        def bn1(s_ref, q_ref, n):
            mu = jnp.sum(s_ref[...], axis=0) / n
            var = jnp.sum(q_ref[...], axis=0) / n - mu * mu
            return mu, lax.rsqrt(var + _EPS)

        def one(y_ref, s_ref, q_ref, k_ref, s2_ref, mx_ref, n, gq, tr):
            mu, inv = bn1(s_ref, q_ref, n)
            mut = jnp.concatenate([mu] * 8).reshape(1, 128)
            invc = jnp.concatenate([inv] * 8).reshape(128, 1)
            t = jnp.maximum(y_ref[...] - mut, 0.0)
            y2 = jnp.dot(t, k_ref[...] * invc,
                         preferred_element_type=jnp.float32)
            s = jnp.sum(y2, axis=0)
            q = jnp.sum(y2 * y2, axis=0)

            @pl.when(i == 0)
            def _init():
                s2_ref[...] = jnp.zeros_like(s2_ref)

            s2_ref[...] += jnp.concatenate([s[None, :], q[None, :]], axis=0)
            c = y2.shape[1]
            mx_ref[...] = jnp.max(y2.reshape(tr // gq, gq, c), axis=1)

        one(y0_ref, s10_ref, sq0_ref, k0_ref, s20_ref, mx0_ref, n0, g0, t0r)
        one(y1_ref, s11_ref, sq1_ref, k1_ref, s21_ref, mx1_ref, n1, g1, t1r)

    sspec = pl.BlockSpec((_NW, _L), lambda i: (0, 0))
    return pl.pallas_call(
        body,
        grid=(grid,),
        in_specs=[pl.BlockSpec((t0r, 128), lambda i: (i, 0)),
                  pl.BlockSpec((t1r, 128), lambda i: (i, 0)),
                  sspec, sspec, sspec, sspec,
                  pl.BlockSpec((128, c0), lambda i: (0, 0)),
                  pl.BlockSpec((128, c1), lambda i: (0, 0))],
        out_specs=[pl.BlockSpec((2, c0), lambda i: (0, 0)),
                   pl.BlockSpec((2, c1), lambda i: (0, 0)),
                   pl.BlockSpec((mq, c0), lambda i: (i, 0)),
                   pl.BlockSpec((mq, c1), lambda i: (i, 0))],
        out_shape=[jax.ShapeDtypeStruct((2, c0), jnp.float32),
                   jax.ShapeDtypeStruct((2, c1), jnp.float32),
                   jax.ShapeDtypeStruct((m, c0), jnp.float32),
                   jax.ShapeDtypeStruct((m, c1), jnp.float32)],
    )(y0p, y1p, s10, sq0, s11, sq1, k0, k1)


def _finalize(mx0, s20, mx1, s21, tile):
    m = mx0.shape[0]
    ns0, ns1 = _NSAMPLES
    n0, n1 = float(m * ns0), float(m * ns1)

    def body(mx0_ref, s20_ref, mx1_ref, s21_ref, out_ref):
        def one(mx_ref, s2_ref, n, c2):
            st = jnp.sum(s2_ref[...].reshape(2, 8, c2), axis=1)
            mu = st[0, :] / n
            var = st[1, :] / n - mu * mu
            inv = lax.rsqrt(var + _EPS)
            mx = jnp.max(mx_ref[...].reshape(tile, 8, c2), axis=1)
            return jnp.maximum((mx - mu) * inv, 0.0)

        a = one(mx0_ref, s20_ref, n0, 16)
        b = one(mx1_ref, s21_ref, n1, 32)
        out_ref[...] = jnp.concatenate([a, b], axis=1)

    return pl.pallas_call(
        body,
        grid=(m // tile,),
        in_specs=[pl.BlockSpec((tile, 128), lambda i: (i, 0)),
                  pl.BlockSpec((2, 128), lambda i: (0, 0)),
                  pl.BlockSpec((tile, 256), lambda i: (i, 0)),
                  pl.BlockSpec((2, 256), lambda i: (0, 0))],
        out_specs=pl.BlockSpec((tile, 48), lambda i: (i, 0)),
        out_shape=jax.ShapeDtypeStruct((m, 48), jnp.float32),
    )(mx0, s20, mx1, s21)


def kernel(xyz, xyz_batch_cnt, new_xyz, new_xyz_batch_cnt, features,
           w0_0, w0_1, w1_0, w1_1):
    B = xyz_batch_cnt.shape[0]
    n_per = xyz.shape[0] // B
    m_per = new_xyz.shape[0] // B
    m = new_xyz.shape[0]
    ns0, ns1 = _NSAMPLES

    u = jnp.concatenate([xyz, features], axis=1)
    h0, h1, o0, o1 = _precompute(u, new_xyz, w0_0.T, w1_0.T,
                                 w0_0[:, :3].T, w1_0[:, :3].T)

    y0f, y1f, s10, sq0, s11, sq1 = _sc_ball_gather(
        xyz[:, 0], xyz[:, 1], xyz[:, 2],
        new_xyz[:, 0], new_xyz[:, 1], new_xyz[:, 2],
        h0, h1, o0.reshape(-1), o1.reshape(-1), n_per, m_per)

    eye8 = jnp.eye(8, dtype=jnp.float32)
    k0 = jnp.kron(eye8, w0_1.T)
    k1 = jnp.kron(eye8, w1_1.T)

    s20, s21, mx0, mx1 = _mlp_fused(
        y0f.reshape(m * ns0 // 8, 128), y1f.reshape(m * ns1 // 8, 128),
        s10.reshape(_NW, _L), sq0.reshape(_NW, _L),
        s11.reshape(_NW, _L), sq1.reshape(_NW, _L), k0, k1, m)
    out = _finalize(mx0, s20, mx1, s21, 2048)
    return (new_xyz, out)


# speedup vs baseline: 86.9568x; 1.0578x over previous
import functools

import jax
import jax.numpy as jnp
from jax import lax
from jax.experimental import pallas as pl
from jax.experimental.pallas import tpu as pltpu
from jax.experimental.pallas import tpu_sc as plsc

_RADII = (0.8, 1.6)
_NSAMPLES = (16, 32)
_EPS = 1e-5

_NC = 2
_NSUB = 16
_NW = _NC * _NSUB
_L = 16
_RC = 128
_GUS = (16, 8)


def _splat(v, dtype=jnp.int32):
    return jnp.full((_L,), v, dtype=dtype)


def _precompute(u, new_xyz, w00t, w10t, wq0, wq1):
    n, ku = u.shape
    m, kq = new_xyz.shape
    grid = 16
    tn, tm = n // grid, m // grid

    def body(u_ref, q_ref, w00_ref, w10_ref, wq0_ref, wq1_ref,
             h0_ref, h1_ref, o0_ref, o1_ref):
        uu = u_ref[...]
        qq = q_ref[...]
        h0_ref[...] = jnp.dot(uu, w00_ref[...], preferred_element_type=jnp.float32)
        h1_ref[...] = jnp.dot(uu, w10_ref[...], preferred_element_type=jnp.float32)
        o0_ref[...] = jnp.dot(qq, wq0_ref[...], preferred_element_type=jnp.float32)
        o1_ref[...] = jnp.dot(qq, wq1_ref[...], preferred_element_type=jnp.float32)

    return pl.pallas_call(
        body,
        grid=(grid,),
        in_specs=[pl.BlockSpec((tn, ku), lambda i: (i, 0)),
                  pl.BlockSpec((tm, kq), lambda i: (i, 0)),
                  pl.BlockSpec((ku, 16), lambda i: (0, 0)),
                  pl.BlockSpec((ku, 16), lambda i: (0, 0)),
                  pl.BlockSpec((kq, 16), lambda i: (0, 0)),
                  pl.BlockSpec((kq, 16), lambda i: (0, 0))],
        out_specs=[pl.BlockSpec((tn, 16), lambda i: (i, 0)),
                   pl.BlockSpec((tn, 16), lambda i: (i, 0)),
                   pl.BlockSpec((tm, 16), lambda i: (i, 0)),
                   pl.BlockSpec((tm, 16), lambda i: (i, 0))],
        out_shape=[jax.ShapeDtypeStruct((n, 16), jnp.float32),
                   jax.ShapeDtypeStruct((n, 16), jnp.float32),
                   jax.ShapeDtypeStruct((m, 16), jnp.float32),
                   jax.ShapeDtypeStruct((m, 16), jnp.float32)],
    )(u, new_xyz, w00t, w10t, wq0, wq1)


def _sc_ball_gather(xs, ys, zs, qx, qy, qz, h0, h1, o0f, o1f, n_per, m_per):
    m = qx.shape[0]
    qw = m // _NW
    ns_max = _NSAMPLES[-1]
    mesh = plsc.VectorSubcoreMesh(core_axis_name="c", subcore_axis_name="s",
                                  num_cores=_NC, num_subcores=_NSUB)

    @functools.partial(
        pl.kernel,
        out_type=[
            jax.ShapeDtypeStruct((m * _NSAMPLES[0], 16), jnp.float32),
            jax.ShapeDtypeStruct((m * _NSAMPLES[1], 16), jnp.float32),
            jax.ShapeDtypeStruct((_NW * _L,), jnp.float32),
            jax.ShapeDtypeStruct((_NW * _L,), jnp.float32),
            jax.ShapeDtypeStruct((_NW * _L,), jnp.float32),
            jax.ShapeDtypeStruct((_NW * _L,), jnp.float32),
        ],
        mesh=mesh,
        compiler_params=pltpu.CompilerParams(needs_layout_passes=False,
                                             use_tc_tiling_on_sc=False),
        scratch_types=[
            pltpu.VMEM((n_per,), jnp.float32),
            pltpu.VMEM((n_per,), jnp.float32),
            pltpu.VMEM((n_per,), jnp.float32),
            pltpu.VMEM((qw,), jnp.float32),
            pltpu.VMEM((qw,), jnp.float32),
            pltpu.VMEM((qw,), jnp.float32),
            pltpu.VMEM((qw * _L,), jnp.float32),
            pltpu.VMEM((qw,), jnp.float32),
            pltpu.VMEM((qw * ns_max,), jnp.int32),
            pltpu.VMEM((_RC, 16), jnp.float32),
            pltpu.VMEM((_RC, 16), jnp.float32),
            pltpu.VMEM((_L,), jnp.float32),
            pltpu.VMEM((_L,), jnp.float32),
            pltpu.SemaphoreType.DMA,
            pltpu.SemaphoreType.DMA,
        ],
    )
    def sc_kernel(xs_h, ys_h, zs_h, qx_h, qy_h, qz_h, h0_h, h1_h, o0_h, o1_h,
                  y0_h, y1_h, s10_h, sq0_h, s11_h, sq1_h,
                  xs_v, ys_v, zs_v, qx_v, qy_v, qz_v, o_v, mask_v, idx_v,
                  rows_v, rows2_v, s1_v, sq_v, sem_a, sem_b):
        wid = lax.axis_index("s") * _NC + lax.axis_index("c")
        qbase = wid * qw
        batch = qbase // m_per
        nbase = batch * n_per

        stage = [
            pltpu.async_copy(xs_h.at[pl.ds(nbase, n_per)], xs_v, sem_a),
            pltpu.async_copy(ys_h.at[pl.ds(nbase, n_per)], ys_v, sem_a),
            pltpu.async_copy(zs_h.at[pl.ds(nbase, n_per)], zs_v, sem_a),
            pltpu.async_copy(qx_h.at[pl.ds(qbase, qw)], qx_v, sem_a),
            pltpu.async_copy(qy_h.at[pl.ds(qbase, qw)], qy_v, sem_a),
            pltpu.async_copy(qz_h.at[pl.ds(qbase, qw)], qz_v, sem_a),
        ]
        for cp in stage:
            cp.wait()

        lanes = lax.broadcasted_iota(jnp.int32, (_L,), 0)
        onesf = jnp.full((_L,), 1.0, jnp.float32)
        zerosf = jnp.zeros((_L,), jnp.float32)

        for scale, (radius, ns) in enumerate(zip(_RADII, _NSAMPLES)):
            r2 = radius * radius
            h_h = (h0_h, h1_h)[scale]
            o_h = (o0_h, o1_h)[scale]
            y_h = (y0_h, y1_h)[scale]
            s_h = (s10_h, s11_h)[scale]
            q_h = (sq0_h, sq1_h)[scale]

            pltpu.sync_copy(o_h.at[pl.ds(qbase * _L, qw * _L)], o_v)

            gu = _GUS[scale]

            def per_query(q, carry, ns=ns, r2=r2, gu=gu):
                qi = _splat(q)
                qxv = plsc.load_gather(qx_v, [qi])
                qyv = plsc.load_gather(qy_v, [qi])
                qzv = plsc.load_gather(qz_v, [qi])

                def cond(c):
                    j, cnt_s, _ = c
                    return jnp.logical_and(cnt_s < ns, j < n_per)

                ones_i = jnp.full((_L,), 1, jnp.int32)
                row_ref = idx_v.at[pl.ds(q * ns, ns)]

                def scan_group(c):
                    j, cnt_s, cntv = c
                    masks, bases = [], []
                    for u in range(gu):
                        jj = j + u * _L
                        dx = xs_v[pl.ds(jj, _L)] - qxv
                        dy = ys_v[pl.ds(jj, _L)] - qyv
                        dz = zs_v[pl.ds(jj, _L)] - qzv
                        d2 = dx * dx + dy * dy + dz * dz
                        within = d2 < r2
                        masks.append(within)
                        bases.append(cntv)
                        cntv = cntv + plsc.all_reduce_population_count(within)
                    cnt_s2 = cntv[0]

                    @pl.when(cnt_s2 > cnt_s)
                    def _emit():
                        for u in range(gu):
                            pos = plsc.cumsum(ones_i, mask=masks[u]) - 1 \
                                + bases[u]
                            smask = jnp.logical_and(masks[u], pos < ns)
                            vals = lanes + ((j + u * _L) + nbase)
                            plsc.store_scatter(row_ref, [pos], vals,
                                               mask=smask)

                    return (j + gu * _L, cnt_s2, cntv)

                fc = lax.while_loop(
                    cond, scan_group,
                    (jnp.int32(0), jnp.int32(0), jnp.zeros((_L,), jnp.int32)))
                cnt = jnp.minimum(fc[1], ns)

                first = plsc.load_gather(idx_v, [_splat(q * ns)])
                cntv = _splat(cnt)
                firstv = jnp.where(cntv > 0, first, _splat(nbase))
                for ch in range(ns // _L):
                    sl = pl.ds(q * ns + ch * _L, _L)
                    keep = (lanes + ch * _L) < cntv
                    idx_v[sl] = jnp.where(keep, idx_v[sl], firstv)
                plsc.store_scatter(mask_v, [_splat(q)],
                                   jnp.where(cntv > 0, onesf, zerosf),
                                   mask=lanes == 0)
                return carry

            lax.fori_loop(0, qw, per_query, 0)

            nsq = _RC // ns
            nch = qw * ns // _RC
            bufs = (rows_v, rows2_v)
            sems = (sem_a, sem_b)

            def _issue(c, b):
                pltpu.async_copy(h_h.at[idx_v.at[pl.ds(c * _RC, _RC)]],
                                 bufs[b], sems[b])

            _issue(0, 0)

            def per_pair(p, carry, ns=ns, nsq=nsq, nch=nch):
                s1, sq = carry
                for b in range(2):
                    c = 2 * p + b
                    buf = bufs[b]

                    @pl.when(c + 1 < nch)
                    def _prefetch():
                        _issue(c + 1, 1 - b)

                    pltpu.make_async_copy(
                        h_h.at[idx_v.at[pl.ds(c * _RC, _RC)]], buf,
                        sems[b]).wait()
                    for qq in range(nsq):
                        qloc = c * nsq + qq
                        ov = o_v[pl.ds(qloc * _L, _L)]
                        mk = plsc.load_gather(mask_v, [_splat(qloc)])
                        for s in range(ns):
                            row = qq * ns + s
                            y = (buf[row] - ov) * mk
                            buf[row] = y
                            s1 = s1 + y
                            sq = sq + y * y
                    pltpu.sync_copy(buf,
                                    y_h.at[pl.ds(qbase * ns + c * _RC, _RC)])
                return (s1, sq)

            s1, sq = lax.fori_loop(0, nch // 2, per_pair,
                                   (zerosf, zerosf))
            s1_v[...] = s1
            sq_v[...] = sq
            pltpu.sync_copy(s1_v, s_h.at[pl.ds(wid * _L, _L)])
            pltpu.sync_copy(sq_v, q_h.at[pl.ds(wid * _L, _L)])

    return sc_kernel(xs, ys, zs, qx, qy, qz, h0, h1, o0f, o1f)


def _mlp_fused(y0p, y1p, s10, sq0, s11, sq1, k0, k1, m):
    r0 = y0p.shape[0]
    r1 = y1p.shape[0]
    c0 = k0.shape[1]
    c1 = k1.shape[1]
    ns0, ns1 = _NSAMPLES
    n0, n1 = float(m * ns0), float(m * ns1)
    g0, g1 = ns0 // 8, ns1 // 8
    grid = 8
    t0r, t1r = r0 // grid, r1 // grid
    mq = m // grid

    def body(y0_ref, y1_ref, s10_ref, sq0_ref, s11_ref, sq1_ref,
             k0_ref, k1_ref, s20_ref, s21_ref, mx0_ref, mx1_ref):
        i = pl.program_id(0)

        def bn1(s_ref, q_ref, n):
            mu = jnp.sum(s_ref[...], axis=0) / n
            var = jnp.sum(q_ref[...], axis=0) / n - mu * mu
            return mu, lax.rsqrt(var + _EPS)

        def one(y_ref, s_ref, q_ref, k_ref, s2_ref, mx_ref, n, gq, tr):
            mu, inv = bn1(s_ref, q_ref, n)
            mut = jnp.concatenate([mu] * 8).reshape(1, 128)
            invc = jnp.concatenate([inv] * 8).reshape(128, 1)
            t = jnp.maximum(y_ref[...] - mut, 0.0)
            y2 = jnp.dot(t, k_ref[...] * invc,
                         preferred_element_type=jnp.float32)
            s = jnp.sum(y2, axis=0)
            q = jnp.sum(y2 * y2, axis=0)

            @pl.when(i == 0)
            def _init():
                s2_ref[...] = jnp.zeros_like(s2_ref)

            s2_ref[...] += jnp.concatenate([s[None, :], q[None, :]], axis=0)
            c = y2.shape[1]
            mx_ref[...] = jnp.max(y2.reshape(tr // gq, gq, c), axis=1)

        one(y0_ref, s10_ref, sq0_ref, k0_ref, s20_ref, mx0_ref, n0, g0, t0r)
        one(y1_ref, s11_ref, sq1_ref, k1_ref, s21_ref, mx1_ref, n1, g1, t1r)

    sspec = pl.BlockSpec((_NW, _L), lambda i: (0, 0))
    return pl.pallas_call(
        body,
        grid=(grid,),
        in_specs=[pl.BlockSpec((t0r, 128), lambda i: (i, 0)),
                  pl.BlockSpec((t1r, 128), lambda i: (i, 0)),
                  sspec, sspec, sspec, sspec,
                  pl.BlockSpec((128, c0), lambda i: (0, 0)),
                  pl.BlockSpec((128, c1), lambda i: (0, 0))],
        out_specs=[pl.BlockSpec((2, c0), lambda i: (0, 0)),
                   pl.BlockSpec((2, c1), lambda i: (0, 0)),
                   pl.BlockSpec((mq, c0), lambda i: (i, 0)),
                   pl.BlockSpec((mq, c1), lambda i: (i, 0))],
        out_shape=[jax.ShapeDtypeStruct((2, c0), jnp.float32),
                   jax.ShapeDtypeStruct((2, c1), jnp.float32),
                   jax.ShapeDtypeStruct((m, c0), jnp.float32),
                   jax.ShapeDtypeStruct((m, c1), jnp.float32)],
    )(y0p, y1p, s10, sq0, s11, sq1, k0, k1)


def _finalize(mx0, s20, mx1, s21, tile):
    m = mx0.shape[0]
    ns0, ns1 = _NSAMPLES
    n0, n1 = float(m * ns0), float(m * ns1)

    def body(mx0_ref, s20_ref, mx1_ref, s21_ref, out_ref):
        def one(mx_ref, s2_ref, n, c2):
            st = jnp.sum(s2_ref[...].reshape(2, 8, c2), axis=1)
            mu = st[0, :] / n
            var = st[1, :] / n - mu * mu
            inv = lax.rsqrt(var + _EPS)
            mx = jnp.max(mx_ref[...].reshape(tile, 8, c2), axis=1)
            return jnp.maximum((mx - mu) * inv, 0.0)

        a = one(mx0_ref, s20_ref, n0, 16)
        b = one(mx1_ref, s21_ref, n1, 32)
        out_ref[...] = jnp.concatenate([a, b], axis=1)

    return pl.pallas_call(
        body,
        grid=(m // tile,),
        in_specs=[pl.BlockSpec((tile, 128), lambda i: (i, 0)),
                  pl.BlockSpec((2, 128), lambda i: (0, 0)),
                  pl.BlockSpec((tile, 256), lambda i: (i, 0)),
                  pl.BlockSpec((2, 256), lambda i: (0, 0))],
        out_specs=pl.BlockSpec((tile, 48), lambda i: (i, 0)),
        out_shape=jax.ShapeDtypeStruct((m, 48), jnp.float32),
    )(mx0, s20, mx1, s21)


def kernel(xyz, xyz_batch_cnt, new_xyz, new_xyz_batch_cnt, features,
           w0_0, w0_1, w1_0, w1_1):
    B = xyz_batch_cnt.shape[0]
    n_per = xyz.shape[0] // B
    m_per = new_xyz.shape[0] // B
    m = new_xyz.shape[0]
    ns0, ns1 = _NSAMPLES

    u = jnp.concatenate([xyz, features], axis=1)
    h0, h1, o0, o1 = _precompute(u, new_xyz, w0_0.T, w1_0.T,
                                 w0_0[:, :3].T, w1_0[:, :3].T)

    y0f, y1f, s10, sq0, s11, sq1 = _sc_ball_gather(
        xyz[:, 0], xyz[:, 1], xyz[:, 2],
        new_xyz[:, 0], new_xyz[:, 1], new_xyz[:, 2],
        h0, h1, o0.reshape(-1), o1.reshape(-1), n_per, m_per)

    eye8 = jnp.eye(8, dtype=jnp.float32)
    k0 = jnp.kron(eye8, w0_1.T)
    k1 = jnp.kron(eye8, w1_1.T)

    s20, s21, mx0, mx1 = _mlp_fused(
        y0f.reshape(m * ns0 // 8, 128), y1f.reshape(m * ns1 // 8, 128),
        s10.reshape(_NW, _L), sq0.reshape(_NW, _L),
        s11.reshape(_NW, _L), sq1.reshape(_NW, _L), k0, k1, m)
    out = _finalize(mx0, s20, mx1, s21, 2048)
    return (new_xyz, out)


# speedup vs baseline: 87.1801x; 1.0026x over previous
import functools

import jax
import jax.numpy as jnp
from jax import lax
from jax.experimental import pallas as pl
from jax.experimental.pallas import tpu as pltpu
from jax.experimental.pallas import tpu_sc as plsc

_RADII = (0.8, 1.6)
_NSAMPLES = (16, 32)
_EPS = 1e-5

_NC = 2
_NSUB = 16
_NW = _NC * _NSUB
_L = 16
_RC = 128
_GUS = (32, 16)


def _splat(v, dtype=jnp.int32):
    return jnp.full((_L,), v, dtype=dtype)


def _precompute(u, new_xyz, w00t, w10t, wq0, wq1):
    n, ku = u.shape
    m, kq = new_xyz.shape
    grid = 16
    tn, tm = n // grid, m // grid

    def body(u_ref, q_ref, w00_ref, w10_ref, wq0_ref, wq1_ref,
             h0_ref, h1_ref, o0_ref, o1_ref):
        uu = u_ref[...]
        qq = q_ref[...]
        h0_ref[...] = jnp.dot(uu, w00_ref[...], preferred_element_type=jnp.float32)
        h1_ref[...] = jnp.dot(uu, w10_ref[...], preferred_element_type=jnp.float32)
        o0_ref[...] = jnp.dot(qq, wq0_ref[...], preferred_element_type=jnp.float32)
        o1_ref[...] = jnp.dot(qq, wq1_ref[...], preferred_element_type=jnp.float32)

    return pl.pallas_call(
        body,
        grid=(grid,),
        in_specs=[pl.BlockSpec((tn, ku), lambda i: (i, 0)),
                  pl.BlockSpec((tm, kq), lambda i: (i, 0)),
                  pl.BlockSpec((ku, 16), lambda i: (0, 0)),
                  pl.BlockSpec((ku, 16), lambda i: (0, 0)),
                  pl.BlockSpec((kq, 16), lambda i: (0, 0)),
                  pl.BlockSpec((kq, 16), lambda i: (0, 0))],
        out_specs=[pl.BlockSpec((tn, 16), lambda i: (i, 0)),
                   pl.BlockSpec((tn, 16), lambda i: (i, 0)),
                   pl.BlockSpec((tm, 16), lambda i: (i, 0)),
                   pl.BlockSpec((tm, 16), lambda i: (i, 0))],
        out_shape=[jax.ShapeDtypeStruct((n, 16), jnp.float32),
                   jax.ShapeDtypeStruct((n, 16), jnp.float32),
                   jax.ShapeDtypeStruct((m, 16), jnp.float32),
                   jax.ShapeDtypeStruct((m, 16), jnp.float32)],
    )(u, new_xyz, w00t, w10t, wq0, wq1)


def _sc_ball_gather(xs, ys, zs, qx, qy, qz, h0, h1, o0f, o1f, n_per, m_per):
    m = qx.shape[0]
    qw = m // _NW
    ns_max = _NSAMPLES[-1]
    mesh = plsc.VectorSubcoreMesh(core_axis_name="c", subcore_axis_name="s",
                                  num_cores=_NC, num_subcores=_NSUB)

    @functools.partial(
        pl.kernel,
        out_type=[
            jax.ShapeDtypeStruct((m * _NSAMPLES[0], 16), jnp.float32),
            jax.ShapeDtypeStruct((m * _NSAMPLES[1], 16), jnp.float32),
            jax.ShapeDtypeStruct((_NW * _L,), jnp.float32),
            jax.ShapeDtypeStruct((_NW * _L,), jnp.float32),
            jax.ShapeDtypeStruct((_NW * _L,), jnp.float32),
            jax.ShapeDtypeStruct((_NW * _L,), jnp.float32),
        ],
        mesh=mesh,
        compiler_params=pltpu.CompilerParams(needs_layout_passes=False,
                                             use_tc_tiling_on_sc=False),
        scratch_types=[
            pltpu.VMEM((n_per,), jnp.float32),
            pltpu.VMEM((n_per,), jnp.float32),
            pltpu.VMEM((n_per,), jnp.float32),
            pltpu.VMEM((qw,), jnp.float32),
            pltpu.VMEM((qw,), jnp.float32),
            pltpu.VMEM((qw,), jnp.float32),
            pltpu.VMEM((qw * _L,), jnp.float32),
            pltpu.VMEM((qw,), jnp.float32),
            pltpu.VMEM((qw * ns_max,), jnp.int32),
            pltpu.VMEM((_RC, 16), jnp.float32),
            pltpu.VMEM((_RC, 16), jnp.float32),
            pltpu.VMEM((_L,), jnp.float32),
            pltpu.VMEM((_L,), jnp.float32),
            pltpu.SemaphoreType.DMA,
            pltpu.SemaphoreType.DMA,
        ],
    )
    def sc_kernel(xs_h, ys_h, zs_h, qx_h, qy_h, qz_h, h0_h, h1_h, o0_h, o1_h,
                  y0_h, y1_h, s10_h, sq0_h, s11_h, sq1_h,
                  xs_v, ys_v, zs_v, qx_v, qy_v, qz_v, o_v, mask_v, idx_v,
                  rows_v, rows2_v, s1_v, sq_v, sem_a, sem_b):
        wid = lax.axis_index("s") * _NC + lax.axis_index("c")
        qbase = wid * qw
        batch = qbase // m_per
        nbase = batch * n_per

        stage = [
            pltpu.async_copy(xs_h.at[pl.ds(nbase, n_per)], xs_v, sem_a),
            pltpu.async_copy(ys_h.at[pl.ds(nbase, n_per)], ys_v, sem_a),
            pltpu.async_copy(zs_h.at[pl.ds(nbase, n_per)], zs_v, sem_a),
            pltpu.async_copy(qx_h.at[pl.ds(qbase, qw)], qx_v, sem_a),
            pltpu.async_copy(qy_h.at[pl.ds(qbase, qw)], qy_v, sem_a),
            pltpu.async_copy(qz_h.at[pl.ds(qbase, qw)], qz_v, sem_a),
        ]
        for cp in stage:
            cp.wait()

        lanes = lax.broadcasted_iota(jnp.int32, (_L,), 0)
        onesf = jnp.full((_L,), 1.0, jnp.float32)
        zerosf = jnp.zeros((_L,), jnp.float32)

        for scale, (radius, ns) in enumerate(zip(_RADII, _NSAMPLES)):
            r2 = radius * radius
            h_h = (h0_h, h1_h)[scale]
            o_h = (o0_h, o1_h)[scale]
            y_h = (y0_h, y1_h)[scale]
            s_h = (s10_h, s11_h)[scale]
            q_h = (sq0_h, sq1_h)[scale]

            pltpu.sync_copy(o_h.at[pl.ds(qbase * _L, qw * _L)], o_v)

            gu = _GUS[scale]

            def per_query(q, carry, ns=ns, r2=r2, gu=gu):
                qi = _splat(q)
                qxv = plsc.load_gather(qx_v, [qi])
                qyv = plsc.load_gather(qy_v, [qi])
                qzv = plsc.load_gather(qz_v, [qi])

                def cond(c):
                    j, cnt_s, _ = c
                    return jnp.logical_and(cnt_s < ns, j < n_per)

                ones_i = jnp.full((_L,), 1, jnp.int32)
                row_ref = idx_v.at[pl.ds(q * ns, ns)]

                def scan_group(c):
                    j, cnt_s, cntv = c
                    masks, bases = [], []
                    for u in range(gu):
                        jj = j + u * _L
                        dx = xs_v[pl.ds(jj, _L)] - qxv
                        dy = ys_v[pl.ds(jj, _L)] - qyv
                        dz = zs_v[pl.ds(jj, _L)] - qzv
                        d2 = dx * dx + dy * dy + dz * dz
                        within = d2 < r2
                        masks.append(within)
                        bases.append(cntv)
                        cntv = cntv + plsc.all_reduce_population_count(within)
                    cnt_s2 = cntv[0]

                    @pl.when(cnt_s2 > cnt_s)
                    def _emit():
                        for u in range(gu):
                            pos = plsc.cumsum(ones_i, mask=masks[u]) - 1 \
                                + bases[u]
                            smask = jnp.logical_and(masks[u], pos < ns)
                            vals = lanes + ((j + u * _L) + nbase)
                            plsc.store_scatter(row_ref, [pos], vals,
                                               mask=smask)

                    return (j + gu * _L, cnt_s2, cntv)

                fc = lax.while_loop(
                    cond, scan_group,
                    (jnp.int32(0), jnp.int32(0), jnp.zeros((_L,), jnp.int32)))
                cnt = jnp.minimum(fc[1], ns)

                first = plsc.load_gather(idx_v, [_splat(q * ns)])
                cntv = _splat(cnt)
                firstv = jnp.where(cntv > 0, first, _splat(nbase))
                for ch in range(ns // _L):
                    sl = pl.ds(q * ns + ch * _L, _L)
                    keep = (lanes + ch * _L) < cntv
                    idx_v[sl] = jnp.where(keep, idx_v[sl], firstv)
                plsc.store_scatter(mask_v, [_splat(q)],
                                   jnp.where(cntv > 0, onesf, zerosf),
                                   mask=lanes == 0)
                return carry

            lax.fori_loop(0, qw, per_query, 0)

            nsq = _RC // ns
            nch = qw * ns // _RC
            bufs = (rows_v, rows2_v)
            sems = (sem_a, sem_b)

            def _issue(c, b):
                pltpu.async_copy(h_h.at[idx_v.at[pl.ds(c * _RC, _RC)]],
                                 bufs[b], sems[b])

            _issue(0, 0)

            def per_pair(p, carry, ns=ns, nsq=nsq, nch=nch):
                s1, sq = carry
                for b in range(2):
                    c = 2 * p + b
                    buf = bufs[b]

                    @pl.when(c + 1 < nch)
                    def _prefetch():
                        _issue(c + 1, 1 - b)

                    pltpu.make_async_copy(
                        h_h.at[idx_v.at[pl.ds(c * _RC, _RC)]], buf,
                        sems[b]).wait()
                    for qq in range(nsq):
                        qloc = c * nsq + qq
                        ov = o_v[pl.ds(qloc * _L, _L)]
                        mk = plsc.load_gather(mask_v, [_splat(qloc)])
                        for s in range(ns):
                            row = qq * ns + s
                            y = (buf[row] - ov) * mk
                            buf[row] = y
                            s1 = s1 + y
                            sq = sq + y * y
                    pltpu.sync_copy(buf,
                                    y_h.at[pl.ds(qbase * ns + c * _RC, _RC)])
                return (s1, sq)

            s1, sq = lax.fori_loop(0, nch // 2, per_pair,
                                   (zerosf, zerosf))
            s1_v[...] = s1
            sq_v[...] = sq
            pltpu.sync_copy(s1_v, s_h.at[pl.ds(wid * _L, _L)])
            pltpu.sync_copy(sq_v, q_h.at[pl.ds(wid * _L, _L)])

    return sc_kernel(xs, ys, zs, qx, qy, qz, h0, h1, o0f, o1f)


def _mlp_fused(y0p, y1p, s10, sq0, s11, sq1, k0, k1, m):
    r0 = y0p.shape[0]
    r1 = y1p.shape[0]
    c0 = k0.shape[1]
    c1 = k1.shape[1]
    ns0, ns1 = _NSAMPLES
    n0, n1 = float(m * ns0), float(m * ns1)
    g0, g1 = ns0 // 8, ns1 // 8
    grid = 8
    t0r, t1r = r0 // grid, r1 // grid
    mq = m // grid

    def body(y0_ref, y1_ref, s10_ref, sq0_ref, s11_ref, sq1_ref,
             k0_ref, k1_ref, s20_ref, s21_ref, mx0_ref, mx1_ref):
        i = pl.program_id(0)

        def bn1(s_ref, q_ref, n):
            mu = jnp.sum(s_ref[...], axis=0) / n
            var = jnp.sum(q_ref[...], axis=0) / n - mu * mu
            return mu, lax.rsqrt(var + _EPS)

        def one(y_ref, s_ref, q_ref, k_ref, s2_ref, mx_ref, n, gq, tr):
            mu, inv = bn1(s_ref, q_ref, n)
            mut = jnp.concatenate([mu] * 8).reshape(1, 128)
            invc = jnp.concatenate([inv] * 8).reshape(128, 1)
            t = jnp.maximum(y_ref[...] - mut, 0.0)
            y2 = jnp.dot(t, k_ref[...] * invc,
                         preferred_element_type=jnp.float32)
            s = jnp.sum(y2, axis=0)
            q = jnp.sum(y2 * y2, axis=0)

            @pl.when(i == 0)
            def _init():
                s2_ref[...] = jnp.zeros_like(s2_ref)

            s2_ref[...] += jnp.concatenate([s[None, :], q[None, :]], axis=0)
            c = y2.shape[1]
            mx_ref[...] = jnp.max(y2.reshape(tr // gq, gq, c), axis=1)

        one(y0_ref, s10_ref, sq0_ref, k0_ref, s20_ref, mx0_ref, n0, g0, t0r)
        one(y1_ref, s11_ref, sq1_ref, k1_ref, s21_ref, mx1_ref, n1, g1, t1r)

    sspec = pl.BlockSpec((_NW, _L), lambda i: (0, 0))
    return pl.pallas_call(
        body,
        grid=(grid,),
        in_specs=[pl.BlockSpec((t0r, 128), lambda i: (i, 0)),
                  pl.BlockSpec((t1r, 128), lambda i: (i, 0)),
                  sspec, sspec, sspec, sspec,
                  pl.BlockSpec((128, c0), lambda i: (0, 0)),
                  pl.BlockSpec((128, c1), lambda i: (0, 0))],
        out_specs=[pl.BlockSpec((2, c0), lambda i: (0, 0)),
                   pl.BlockSpec((2, c1), lambda i: (0, 0)),
                   pl.BlockSpec((mq, c0), lambda i: (i, 0)),
                   pl.BlockSpec((mq, c1), lambda i: (i, 0))],
        out_shape=[jax.ShapeDtypeStruct((2, c0), jnp.float32),
                   jax.ShapeDtypeStruct((2, c1), jnp.float32),
                   jax.ShapeDtypeStruct((m, c0), jnp.float32),
                   jax.ShapeDtypeStruct((m, c1), jnp.float32)],
    )(y0p, y1p, s10, sq0, s11, sq1, k0, k1)


def _finalize(mx0, s20, mx1, s21, tile):
    m = mx0.shape[0]
    ns0, ns1 = _NSAMPLES
    n0, n1 = float(m * ns0), float(m * ns1)

    def body(mx0_ref, s20_ref, mx1_ref, s21_ref, out_ref):
        def one(mx_ref, s2_ref, n, c2):
            st = jnp.sum(s2_ref[...].reshape(2, 8, c2), axis=1)
            mu = st[0, :] / n
            var = st[1, :] / n - mu * mu
            inv = lax.rsqrt(var + _EPS)
            mx = jnp.max(mx_ref[...].reshape(tile, 8, c2), axis=1)
            return jnp.maximum((mx - mu) * inv, 0.0)

        a = one(mx0_ref, s20_ref, n0, 16)
        b = one(mx1_ref, s21_ref, n1, 32)
        out_ref[...] = jnp.concatenate([a, b], axis=1)

    return pl.pallas_call(
        body,
        grid=(m // tile,),
        in_specs=[pl.BlockSpec((tile, 128), lambda i: (i, 0)),
                  pl.BlockSpec((2, 128), lambda i: (0, 0)),
                  pl.BlockSpec((tile, 256), lambda i: (i, 0)),
                  pl.BlockSpec((2, 256), lambda i: (0, 0))],
        out_specs=pl.BlockSpec((tile, 48), lambda i: (i, 0)),
        out_shape=jax.ShapeDtypeStruct((m, 48), jnp.float32),
    )(mx0, s20, mx1, s21)


def kernel(xyz, xyz_batch_cnt, new_xyz, new_xyz_batch_cnt, features,
           w0_0, w0_1, w1_0, w1_1):
    B = xyz_batch_cnt.shape[0]
    n_per = xyz.shape[0] // B
    m_per = new_xyz.shape[0] // B
    m = new_xyz.shape[0]
    ns0, ns1 = _NSAMPLES

    u = jnp.concatenate([xyz, features], axis=1)
    h0, h1, o0, o1 = _precompute(u, new_xyz, w0_0.T, w1_0.T,
                                 w0_0[:, :3].T, w1_0[:, :3].T)

    y0f, y1f, s10, sq0, s11, sq1 = _sc_ball_gather(
        xyz[:, 0], xyz[:, 1], xyz[:, 2],
        new_xyz[:, 0], new_xyz[:, 1], new_xyz[:, 2],
        h0, h1, o0.reshape(-1), o1.reshape(-1), n_per, m_per)

    eye8 = jnp.eye(8, dtype=jnp.float32)
    k0 = jnp.kron(eye8, w0_1.T)
    k1 = jnp.kron(eye8, w1_1.T)

    s20, s21, mx0, mx1 = _mlp_fused(
        y0f.reshape(m * ns0 // 8, 128), y1f.reshape(m * ns1 // 8, 128),
        s10.reshape(_NW, _L), sq0.reshape(_NW, _L),
        s11.reshape(_NW, _L), sq1.reshape(_NW, _L), k0, k1, m)
    out = _finalize(mx0, s20, mx1, s21, 2048)
    return (new_xyz, out)
